# Initial kernel scaffold; baseline (speedup 1.0000x reference)
import jax
import jax.numpy as jnp
from jax.experimental import pallas as pl


def kernel(x, edge_index, W1, att_src1, att_dst1, bias1, W2, att_src2, att_dst2, bias2, W_skip, b_skip, W_final, b_final, ln1_g, ln1_b, ln2_g, ln2_b):
    raise NotImplementedError("write your pallas kernel here")


# speedup vs baseline: 42.8270x; 42.8270x over previous
import jax
import jax.numpy as jnp
from jax import lax
from jax.experimental import pallas as pl
from jax.experimental.pallas import tpu as pltpu
from jax.experimental.pallas import tpu_sc as plsc

_N = 10000
_E = 320000
_D = 128
_HPH = 64
_B = 128
_NC = 2
_NS = 16
_BPC = _E // _NC // _B
_BLK_LO = _BPC // _NS
_BLK_REM = _BPC % _NS
_OPT = _N // 8 // _NS
_OREM = (_N // 8) % _NS
_CH = 104
_G16 = _N // 16
_GPT = _G16 // _NS
_GREM = _G16 % _NS
_WPT = _GPT * 16
_f32 = jnp.float32


def _lrelu(a):
    return jnp.where(a >= 0.0, a, 0.2 * a)


_mesh = plsc.VectorSubcoreMesh(core_axis_name="c", subcore_axis_name="s")


def _make_edge_pass(two_heads):
    nhd = 2 if two_heads else 1
    scratch = [
        pltpu.VMEM((_B,), jnp.int32),
        pltpu.VMEM((_B,), jnp.int32),
        pltpu.VMEM((_B, _D), _f32),
        pltpu.VMEM((640,), _f32),
    ]
    scratch += [pltpu.VMEM((_B,), _f32) for _ in range(2 * nhd)]
    scratch += [pltpu.VMEM((_B,), _f32) for _ in range(nhd)]
    scratch += [pltpu.VMEM_SHARED((_N,), _f32) for _ in range(nhd)]
    scratch += [
        pltpu.VMEM_SHARED((_N, _D), _f32),
        pltpu.SemaphoreType.DMA,
    ]

    def body(*refs):
        it = iter(refs)
        h_hbm = next(it)
        src_h = next(it)
        dst_h = next(it)
        as0_h = next(it)
        ad0_h = next(it)
        as1_h = next(it) if two_heads else None
        ad1_h = next(it) if two_heads else None
        feat_o = next(it)
        den_o = next(it)
        src_v = next(it)
        dst_v = next(it)
        rows_v = next(it)
        z_v = next(it)
        a0s_v = next(it)
        a0d_v = next(it)
        a1s_v = next(it) if two_heads else None
        a1d_v = next(it) if two_heads else None
        ex0_v = next(it)
        ex1_v = next(it) if two_heads else None
        den0_sh = next(it)
        den1_sh = next(it) if two_heads else None
        acc = next(it)
        sem = next(it)

        cid = lax.axis_index("c")
        sid = lax.axis_index("s")
        zero = jnp.zeros((16,), _f32)

        for i in range(640 // 16):
            z_v[pl.ds(i * 16, 16)] = zero

        def _zrow(r, c_):
            for c in range(_D // 16):
                rows_v[r, pl.ds(c * 16, 16)] = zero
            return c_

        lax.fori_loop(0, _CH, _zrow, 0)

        base_row = pl.multiple_of(8 * (_OPT * sid + jnp.minimum(sid, _OREM)), 8)
        for k in range(_OPT * 8 // _CH):
            pltpu.sync_copy(rows_v.at[pl.ds(0, _CH)],
                            acc.at[pl.ds(base_row + k * _CH, _CH)])

        @pl.when(sid < _OREM)
        def _init_tail():
            pltpu.sync_copy(rows_v.at[pl.ds(0, 8)],
                            acc.at[pl.ds(base_row + _OPT * 8, 8)])

        base_w = pl.multiple_of(16 * (_GPT * sid + jnp.minimum(sid, _GREM)), 16)
        pltpu.sync_copy(z_v.at[pl.ds(0, _WPT)],
                        den0_sh.at[pl.ds(base_w, _WPT)])
        if two_heads:
            pltpu.sync_copy(z_v.at[pl.ds(0, _WPT)],
                            den1_sh.at[pl.ds(base_w, _WPT)])

        @pl.when(sid < _GREM)
        def _init_tail_w():
            pltpu.sync_copy(z_v.at[pl.ds(0, 16)],
                            den0_sh.at[pl.ds(base_w + _WPT, 16)])
            if two_heads:
                pltpu.sync_copy(z_v.at[pl.ds(0, 16)],
                                den1_sh.at[pl.ds(base_w + _WPT, 16)])

        plsc.subcore_barrier()

        nblk = _BLK_LO + jnp.where(sid < _BLK_REM, 1, 0)
        blk0 = cid * _BPC + _BLK_LO * sid + jnp.minimum(sid, _BLK_REM)

        def _block(b, carry):
            base = (blk0 + b) * _B
            pltpu.sync_copy(src_h.at[pl.ds(base, _B)], src_v)
            pltpu.sync_copy(dst_h.at[pl.ds(base, _B)], dst_v)
            pltpu.async_copy(as0_h.at[src_v], a0s_v, sem).wait()
            pltpu.async_copy(ad0_h.at[dst_v], a0d_v, sem).wait()
            if two_heads:
                pltpu.async_copy(as1_h.at[src_v], a1s_v, sem).wait()
                pltpu.async_copy(ad1_h.at[dst_v], a1d_v, sem).wait()
            pltpu.async_copy(h_hbm.at[src_v], rows_v, sem).wait()

            def _grp(g, c_):
                off = g * 16
                a0 = a0s_v[pl.ds(off, 16)] + a0d_v[pl.ds(off, 16)]
                ex0 = jnp.exp(_lrelu(a0))
                ex0_v[pl.ds(off, 16)] = ex0
                if two_heads:
                    a1 = a1s_v[pl.ds(off, 16)] + a1d_v[pl.ds(off, 16)]
                    ex1 = jnp.exp(_lrelu(a1))
                    ex1_v[pl.ds(off, 16)] = ex1
                for j in range(16):
                    e = off + j
                    idx = jnp.full((16,), e, jnp.int32)
                    v0 = plsc.load_gather(ex0_v, [idx])
                    v1 = plsc.load_gather(ex1_v, [idx]) if two_heads else v0
                    for c in range(4):
                        rows_v[e, pl.ds(c * 16, 16)] = rows_v[e, pl.ds(c * 16, 16)] * v0
                    for c in range(4, 8):
                        rows_v[e, pl.ds(c * 16, 16)] = rows_v[e, pl.ds(c * 16, 16)] * v1
                return c_

            lax.fori_loop(0, _B // 16, _grp, 0)
            pltpu.sync_copy(rows_v, acc.at[dst_v], add=True)
            pltpu.sync_copy(ex0_v, den0_sh.at[dst_v], add=True)
            if two_heads:
                pltpu.sync_copy(ex1_v, den1_sh.at[dst_v], add=True)
            return carry

        lax.fori_loop(0, nblk, _block, 0)

        plsc.subcore_barrier()
        for k in range(_OPT * 8 // _CH):
            pltpu.sync_copy(acc.at[pl.ds(base_row + k * _CH, _CH)],
                            feat_o.at[pl.ds(cid * _N + base_row + k * _CH, _CH)])
        pltpu.sync_copy(den0_sh.at[pl.ds(base_w, _WPT)], z_v.at[pl.ds(0, _WPT)])
        pltpu.sync_copy(z_v.at[pl.ds(0, _WPT)],
                        den_o.at[pl.ds(cid * _N + base_w, _WPT)])
        if two_heads:
            pltpu.sync_copy(den1_sh.at[pl.ds(base_w, _WPT)],
                            z_v.at[pl.ds(0, _WPT)])
            pltpu.sync_copy(z_v.at[pl.ds(0, _WPT)],
                            den_o.at[pl.ds((_NC + cid) * _N + base_w, _WPT)])

        @pl.when(sid < _OREM)
        def _drain_tail():
            pltpu.sync_copy(acc.at[pl.ds(base_row + _OPT * 8, 8)],
                            feat_o.at[pl.ds(cid * _N + base_row + _OPT * 8, 8)])

        @pl.when(sid < _GREM)
        def _drain_tail_w():
            pltpu.sync_copy(den0_sh.at[pl.ds(base_w + _WPT, 16)],
                            z_v.at[pl.ds(0, 16)])
            pltpu.sync_copy(z_v.at[pl.ds(0, 16)],
                            den_o.at[pl.ds(cid * _N + base_w + _WPT, 16)])
            if two_heads:
                pltpu.sync_copy(den1_sh.at[pl.ds(base_w + _WPT, 16)],
                                z_v.at[pl.ds(0, 16)])
                pltpu.sync_copy(z_v.at[pl.ds(0, 16)],
                                den_o.at[pl.ds((_NC + cid) * _N + base_w + _WPT, 16)])

    return pl.kernel(
        body,
        out_type=(
            jax.ShapeDtypeStruct((_NC * _N, _D), _f32),
            jax.ShapeDtypeStruct((nhd * _NC * _N,), _f32),
        ),
        mesh=_mesh,
        scratch_types=scratch,
        compiler_params=pltpu.CompilerParams(needs_layout_passes=False),
    )


_edge_pass2 = _make_edge_pass(True)
_edge_pass1 = _make_edge_pass(False)


def _den_col(dp, lo):
    return lax.dot_general(dp[lo:lo + _NC, :], jnp.ones((_NC, 1), _f32),
                           (((0,), (0,)), ((), ())),
                           preferred_element_type=_f32)


def _stage_a(x_ref, w1_ref, as1_ref, ad1_ref, wsk_ref, bsk_ref,
             h_ref, as0_o, as1_o, ad0_o, ad1_o, xskip_ref):
    x = x_ref[...]
    h = jnp.dot(x, w1_ref[...], preferred_element_type=_f32)
    att_s = as1_ref[...]
    att_d = ad1_ref[...]
    h0 = h[:, :_HPH]
    h1 = h[:, _HPH:]
    as0_o[...] = jnp.sum(h0 * att_s[0:1, :], axis=1, keepdims=True)
    as1_o[...] = jnp.sum(h1 * att_s[1:2, :], axis=1, keepdims=True)
    ad0_o[...] = jnp.sum(h0 * att_d[0:1, :], axis=1, keepdims=True)
    ad1_o[...] = jnp.sum(h1 * att_d[1:2, :], axis=1, keepdims=True)
    h_ref[...] = h
    xskip_ref[...] = (jnp.dot(x, wsk_ref[...], preferred_element_type=_f32)
                      + bsk_ref[...][None, :])


def _stage_c(part_ref, dp_ref, h1_ref, as0_ref, as1_ref, ad0_ref, ad1_ref,
             xskip_ref, b1_ref, g1_ref, bb1_ref, w2_ref, as2_ref, ad2_ref,
             wf_ref, bf_ref,
             h2_ref, a2s_ref, a2d_ref, xfin_ref):
    acc = part_ref[0] + part_ref[1]
    dp = dp_ref[0]
    e0 = jnp.exp(_lrelu(as0_ref[...] + ad0_ref[...]))
    e1 = jnp.exp(_lrelu(as1_ref[...] + ad1_ref[...]))
    h = h1_ref[...]
    num0 = acc[:, :_HPH] + h[:, :_HPH] * e0
    num1 = acc[:, _HPH:_D] + h[:, _HPH:_D] * e1
    den0 = _den_col(dp, 0) + e0
    den1 = _den_col(dp, _NC) + e1
    gat = jnp.concatenate([num0 / (den0 + 1e-16), num1 / (den1 + 1e-16)],
                          axis=1) + b1_ref[...][None, :]
    xhop = jnp.where(gat > 0.0, gat, jnp.exp(jnp.minimum(gat, 0.0)) - 1.0)
    xcomb = xhop + xskip_ref[...]
    m = jnp.mean(xcomb, axis=1, keepdims=True)
    v = jnp.mean((xcomb - m) ** 2, axis=1, keepdims=True)
    first = ((xcomb - m) / jnp.sqrt(v + 1e-5) * g1_ref[...][None, :]
             + bb1_ref[...][None, :])
    h2 = jnp.dot(first, w2_ref[...], preferred_element_type=_f32)
    a2s_ref[...] = jnp.sum(h2 * as2_ref[...], axis=1, keepdims=True)
    a2d_ref[...] = jnp.sum(h2 * ad2_ref[...], axis=1, keepdims=True)
    h2_ref[...] = h2
    xfin_ref[...] = (jnp.dot(first, wf_ref[...], preferred_element_type=_f32)
                     + bf_ref[...][None, :])


def _stage_e(part_ref, dp_ref, h2_ref, a2s_ref, a2d_ref, xfin_ref,
             b2_ref, g2_ref, bb2_ref, out_ref):
    acc = part_ref[0:_N, :] + part_ref[_N:2 * _N, :]
    e2 = jnp.exp(_lrelu(a2s_ref[...] + a2d_ref[...]))
    num = acc + h2_ref[...] * e2
    den = _den_col(dp_ref[0], 0) + e2
    x2 = num / (den + 1e-16) + b2_ref[...][None, :]
    y = x2 + xfin_ref[...]
    m = jnp.mean(y, axis=1, keepdims=True)
    v = jnp.mean((y - m) ** 2, axis=1, keepdims=True)
    out_ref[...] = ((y - m) / jnp.sqrt(v + 1e-5) * g2_ref[...][None, :]
                    + bb2_ref[...][None, :])


_stage_a_call = pl.pallas_call(
    _stage_a,
    out_shape=[
        jax.ShapeDtypeStruct((_N, _D), _f32),
        jax.ShapeDtypeStruct((_N, 1), _f32),
        jax.ShapeDtypeStruct((_N, 1), _f32),
        jax.ShapeDtypeStruct((_N, 1), _f32),
        jax.ShapeDtypeStruct((_N, 1), _f32),
        jax.ShapeDtypeStruct((_N, _D), _f32),
    ],
)
_BS = 2000
_stage_c_call = pl.pallas_call(
    _stage_c,
    grid=(_N // _BS,),
    in_specs=[
        pl.BlockSpec((2, _BS, _D), lambda i: (0, i, 0)),
        pl.BlockSpec((1, 2 * _NC, _BS), lambda i: (i, 0, 0)),
        pl.BlockSpec((_BS, _D), lambda i: (i, 0)),
        pl.BlockSpec((_BS, 1), lambda i: (i, 0)),
        pl.BlockSpec((_BS, 1), lambda i: (i, 0)),
        pl.BlockSpec((_BS, 1), lambda i: (i, 0)),
        pl.BlockSpec((_BS, 1), lambda i: (i, 0)),
        pl.BlockSpec((_BS, _D), lambda i: (i, 0)),
        pl.BlockSpec((_D,), lambda i: (0,)),
        pl.BlockSpec((_D,), lambda i: (0,)),
        pl.BlockSpec((_D,), lambda i: (0,)),
        pl.BlockSpec((_D, _D), lambda i: (0, 0)),
        pl.BlockSpec((1, _D), lambda i: (0, 0)),
        pl.BlockSpec((1, _D), lambda i: (0, 0)),
        pl.BlockSpec((_D, _D), lambda i: (0, 0)),
        pl.BlockSpec((_D,), lambda i: (0,)),
    ],
    out_specs=[
        pl.BlockSpec((_BS, _D), lambda i: (i, 0)),
        pl.BlockSpec((_BS, 1), lambda i: (i, 0)),
        pl.BlockSpec((_BS, 1), lambda i: (i, 0)),
        pl.BlockSpec((_BS, _D), lambda i: (i, 0)),
    ],
    out_shape=[
        jax.ShapeDtypeStruct((_N, _D), _f32),
        jax.ShapeDtypeStruct((_N, 1), _f32),
        jax.ShapeDtypeStruct((_N, 1), _f32),
        jax.ShapeDtypeStruct((_N, _D), _f32),
    ],
)
_stage_e_call = pl.pallas_call(
    _stage_e,
    out_shape=jax.ShapeDtypeStruct((_N, _D), _f32),
)


@jax.jit
def _run(x, edge_index, W1, att_src1, att_dst1, bias1, W2, att_src2,
         att_dst2, bias2, W_skip, b_skip, W_final, b_final,
         ln1_g, ln1_b, ln2_g, ln2_b):
    src = edge_index[0]
    dst = edge_index[1]
    h1, as0, as1, ad0, ad1, xskip = _stage_a_call(
        x, W1, att_src1, att_dst1, W_skip, b_skip)
    part1, dp1 = _edge_pass2(h1, src, dst,
                             as0.reshape(_N), ad0.reshape(_N),
                             as1.reshape(_N), ad1.reshape(_N))
    dp1 = jnp.swapaxes(dp1.reshape(2 * _NC, _N // _BS, _BS), 0, 1)
    part1 = part1.reshape(2, _N, _D)
    h2, a2s, a2d, xfin = _stage_c_call(
        part1, dp1, h1, as0, as1, ad0, ad1, xskip, bias1, ln1_g, ln1_b,
        W2, att_src2, att_dst2, W_final, b_final)
    part2, dp2 = _edge_pass1(h2, src, dst, a2s.reshape(_N), a2d.reshape(_N))
    dp2 = dp2.reshape(1, _NC, _N)
    return _stage_e_call(part2, dp2, h2, a2s, a2d, xfin, bias2, ln2_g, ln2_b)


def kernel(x, edge_index, W1, att_src1, att_dst1, bias1, W2, att_src2,
           att_dst2, bias2, W_skip, b_skip, W_final, b_final,
           ln1_g, ln1_b, ln2_g, ln2_b):
    return _run(x, edge_index, W1, att_src1, att_dst1, bias1, W2, att_src2,
                att_dst2, bias2, W_skip, b_skip, W_final, b_final,
                ln1_g, ln1_b, ln2_g, ln2_b)


# speedup vs baseline: 79.8563x; 1.8646x over previous
import jax
import jax.numpy as jnp
from jax import lax
from jax.experimental import pallas as pl
from jax.experimental.pallas import tpu as pltpu
from jax.experimental.pallas import tpu_sc as plsc

_N = 10000
_E = 320000
_D = 128
_HPH = 64
_B = 128
_NC = 2
_NS = 16
_BPC = _E // _NC // _B
_BLK_LO = _BPC // _NS
_BLK_REM = _BPC % _NS
_OPT = _N // 8 // _NS
_OREM = (_N // 8) % _NS
_CH = 104
_G16 = _N // 16
_GPT = _G16 // _NS
_GREM = _G16 % _NS
_WPT = _GPT * 16
_f32 = jnp.float32


def _lrelu(a):
    return jnp.where(a >= 0.0, a, 0.2 * a)


_mesh = plsc.VectorSubcoreMesh(core_axis_name="c", subcore_axis_name="s")


def _make_edge_pass(two_heads):
    nhd = 2 if two_heads else 1
    bufset = (
        [pltpu.VMEM((_B,), jnp.int32) for _ in range(2)]
        + [pltpu.VMEM((_B, _D), _f32)]
        + [pltpu.VMEM((_B,), _f32) for _ in range(2 * nhd)]
    )
    scratch = bufset + bufset + [
        pltpu.VMEM((640,), _f32),
    ]
    scratch += [pltpu.VMEM((_B,), _f32) for _ in range(nhd)]
    scratch += [pltpu.VMEM_SHARED((_N,), _f32) for _ in range(nhd)]
    scratch += [
        pltpu.VMEM_SHARED((_N, _D), _f32),
        pltpu.SemaphoreType.DMA,
        pltpu.SemaphoreType.DMA,
    ]

    def body(*refs):
        it = iter(refs)
        h_hbm = next(it)
        src_h = next(it)
        dst_h = next(it)
        as0_h = next(it)
        ad0_h = next(it)
        as1_h = next(it) if two_heads else None
        ad1_h = next(it) if two_heads else None
        feat_o = next(it)
        den_o = next(it)

        def _take_set():
            s = {}
            s["src"] = next(it)
            s["dst"] = next(it)
            s["rows"] = next(it)
            s["a0s"] = next(it)
            s["a0d"] = next(it)
            s["a1s"] = next(it) if two_heads else None
            s["a1d"] = next(it) if two_heads else None
            return s

        bufA = _take_set()
        bufB = _take_set()
        z_v = next(it)
        ex0_v = next(it)
        ex1_v = next(it) if two_heads else None
        den0_sh = next(it)
        den1_sh = next(it) if two_heads else None
        acc = next(it)
        semA = next(it)
        semB = next(it)
        rows_v = bufA["rows"]

        cid = lax.axis_index("c")
        sid = lax.axis_index("s")
        zero = jnp.zeros((16,), _f32)

        for i in range(640 // 16):
            z_v[pl.ds(i * 16, 16)] = zero

        def _zrow(r, c_):
            for c in range(_D // 16):
                rows_v[r, pl.ds(c * 16, 16)] = zero
            return c_

        lax.fori_loop(0, _CH, _zrow, 0)

        base_row = pl.multiple_of(8 * (_OPT * sid + jnp.minimum(sid, _OREM)), 8)
        for k in range(_OPT * 8 // _CH):
            pltpu.sync_copy(rows_v.at[pl.ds(0, _CH)],
                            acc.at[pl.ds(base_row + k * _CH, _CH)])

        @pl.when(sid < _OREM)
        def _init_tail():
            pltpu.sync_copy(rows_v.at[pl.ds(0, 8)],
                            acc.at[pl.ds(base_row + _OPT * 8, 8)])

        base_w = pl.multiple_of(16 * (_GPT * sid + jnp.minimum(sid, _GREM)), 16)
        pltpu.sync_copy(z_v.at[pl.ds(0, _WPT)],
                        den0_sh.at[pl.ds(base_w, _WPT)])
        if two_heads:
            pltpu.sync_copy(z_v.at[pl.ds(0, _WPT)],
                            den1_sh.at[pl.ds(base_w, _WPT)])

        @pl.when(sid < _GREM)
        def _init_tail_w():
            pltpu.sync_copy(z_v.at[pl.ds(0, 16)],
                            den0_sh.at[pl.ds(base_w + _WPT, 16)])
            if two_heads:
                pltpu.sync_copy(z_v.at[pl.ds(0, 16)],
                                den1_sh.at[pl.ds(base_w + _WPT, 16)])

        plsc.subcore_barrier()

        blk0 = cid * _BPC + _BLK_LO * sid + jnp.minimum(sid, _BLK_REM)
        last_blk = _E // _B - 1

        def _fire(blk, buf, sem):
            base = blk * _B
            pltpu.sync_copy(src_h.at[pl.ds(base, _B)], buf["src"])
            pltpu.sync_copy(dst_h.at[pl.ds(base, _B)], buf["dst"])
            pltpu.async_copy(as0_h.at[buf["src"]], buf["a0s"], sem)
            pltpu.async_copy(ad0_h.at[buf["dst"]], buf["a0d"], sem)
            if two_heads:
                pltpu.async_copy(as1_h.at[buf["src"]], buf["a1s"], sem)
                pltpu.async_copy(ad1_h.at[buf["dst"]], buf["a1d"], sem)
            pltpu.async_copy(h_hbm.at[buf["src"]], buf["rows"], sem)

        def _drain(buf, sem):
            pltpu.make_async_copy(as0_h.at[buf["src"]], buf["a0s"], sem).wait()
            pltpu.make_async_copy(ad0_h.at[buf["dst"]], buf["a0d"], sem).wait()
            if two_heads:
                pltpu.make_async_copy(as1_h.at[buf["src"]], buf["a1s"], sem).wait()
                pltpu.make_async_copy(ad1_h.at[buf["dst"]], buf["a1d"], sem).wait()
            pltpu.make_async_copy(h_hbm.at[buf["src"]], buf["rows"], sem).wait()

        def _consume(buf):
            rows = buf["rows"]

            def _grp(g, c_):
                off = g * 16
                a0 = buf["a0s"][pl.ds(off, 16)] + buf["a0d"][pl.ds(off, 16)]
                ex0 = jnp.exp(_lrelu(a0))
                ex0_v[pl.ds(off, 16)] = ex0
                if two_heads:
                    a1 = buf["a1s"][pl.ds(off, 16)] + buf["a1d"][pl.ds(off, 16)]
                    ex1 = jnp.exp(_lrelu(a1))
                    ex1_v[pl.ds(off, 16)] = ex1
                for j in range(16):
                    e = off + j
                    idx = jnp.full((16,), e, jnp.int32)
                    v0 = plsc.load_gather(ex0_v, [idx])
                    v1 = plsc.load_gather(ex1_v, [idx]) if two_heads else v0
                    for c in range(4):
                        rows[e, pl.ds(c * 16, 16)] = rows[e, pl.ds(c * 16, 16)] * v0
                    for c in range(4, 8):
                        rows[e, pl.ds(c * 16, 16)] = rows[e, pl.ds(c * 16, 16)] * v1
                return c_

            lax.fori_loop(0, _B // 16, _grp, 0)
            pltpu.sync_copy(rows, acc.at[buf["dst"]], add=True)
            pltpu.sync_copy(ex0_v, den0_sh.at[buf["dst"]], add=True)
            if two_heads:
                pltpu.sync_copy(ex1_v, den1_sh.at[buf["dst"]], add=True)

        _fire(blk0, bufA, semA)

        def _pair(k, carry):
            bA = blk0 + 2 * k
            _fire(bA + 1, bufB, semB)
            _drain(bufA, semA)
            _consume(bufA)
            _fire(jnp.minimum(bA + 2, last_blk), bufA, semA)
            _drain(bufB, semB)
            _consume(bufB)
            return carry

        lax.fori_loop(0, _BLK_LO // 2, _pair, 0)

        _drain(bufA, semA)

        @pl.when(sid < _BLK_REM)
        def _tail_block():
            _consume(bufA)

        plsc.subcore_barrier()
        for k in range(_OPT * 8 // _CH):
            pltpu.sync_copy(acc.at[pl.ds(base_row + k * _CH, _CH)],
                            feat_o.at[pl.ds(cid * _N + base_row + k * _CH, _CH)])
        pltpu.sync_copy(den0_sh.at[pl.ds(base_w, _WPT)], z_v.at[pl.ds(0, _WPT)])
        pltpu.sync_copy(z_v.at[pl.ds(0, _WPT)],
                        den_o.at[pl.ds(cid * _N + base_w, _WPT)])
        if two_heads:
            pltpu.sync_copy(den1_sh.at[pl.ds(base_w, _WPT)],
                            z_v.at[pl.ds(0, _WPT)])
            pltpu.sync_copy(z_v.at[pl.ds(0, _WPT)],
                            den_o.at[pl.ds((_NC + cid) * _N + base_w, _WPT)])

        @pl.when(sid < _OREM)
        def _drain_tail():
            pltpu.sync_copy(acc.at[pl.ds(base_row + _OPT * 8, 8)],
                            feat_o.at[pl.ds(cid * _N + base_row + _OPT * 8, 8)])

        @pl.when(sid < _GREM)
        def _drain_tail_w():
            pltpu.sync_copy(den0_sh.at[pl.ds(base_w + _WPT, 16)],
                            z_v.at[pl.ds(0, 16)])
            pltpu.sync_copy(z_v.at[pl.ds(0, 16)],
                            den_o.at[pl.ds(cid * _N + base_w + _WPT, 16)])
            if two_heads:
                pltpu.sync_copy(den1_sh.at[pl.ds(base_w + _WPT, 16)],
                                z_v.at[pl.ds(0, 16)])
                pltpu.sync_copy(z_v.at[pl.ds(0, 16)],
                                den_o.at[pl.ds((_NC + cid) * _N + base_w + _WPT, 16)])

    return pl.kernel(
        body,
        out_type=(
            jax.ShapeDtypeStruct((_NC * _N, _D), _f32),
            jax.ShapeDtypeStruct((nhd * _NC * _N,), _f32),
        ),
        mesh=_mesh,
        scratch_types=scratch,
        compiler_params=pltpu.CompilerParams(needs_layout_passes=False),
    )


_edge_pass2 = _make_edge_pass(True)
_edge_pass1 = _make_edge_pass(False)


def _den_col(dp, lo):
    return lax.dot_general(dp[lo:lo + _NC, :], jnp.ones((_NC, 1), _f32),
                           (((0,), (0,)), ((), ())),
                           preferred_element_type=_f32)


def _stage_a(x_ref, w1_ref, as1_ref, ad1_ref, wsk_ref, bsk_ref,
             h_ref, as0_o, as1_o, ad0_o, ad1_o, xskip_ref):
    x = x_ref[...]
    h = jnp.dot(x, w1_ref[...], preferred_element_type=_f32)
    att_s = as1_ref[...]
    att_d = ad1_ref[...]
    h0 = h[:, :_HPH]
    h1 = h[:, _HPH:]
    as0_o[...] = jnp.sum(h0 * att_s[0:1, :], axis=1, keepdims=True)
    as1_o[...] = jnp.sum(h1 * att_s[1:2, :], axis=1, keepdims=True)
    ad0_o[...] = jnp.sum(h0 * att_d[0:1, :], axis=1, keepdims=True)
    ad1_o[...] = jnp.sum(h1 * att_d[1:2, :], axis=1, keepdims=True)
    h_ref[...] = h
    xskip_ref[...] = (jnp.dot(x, wsk_ref[...], preferred_element_type=_f32)
                      + bsk_ref[...][None, :])


def _stage_c(part_ref, dp_ref, h1_ref, as0_ref, as1_ref, ad0_ref, ad1_ref,
             xskip_ref, b1_ref, g1_ref, bb1_ref, w2_ref, as2_ref, ad2_ref,
             wf_ref, bf_ref,
             h2_ref, a2s_ref, a2d_ref, xfin_ref):
    acc = part_ref[0] + part_ref[1]
    dp = dp_ref[0]
    e0 = jnp.exp(_lrelu(as0_ref[...] + ad0_ref[...]))
    e1 = jnp.exp(_lrelu(as1_ref[...] + ad1_ref[...]))
    h = h1_ref[...]
    num0 = acc[:, :_HPH] + h[:, :_HPH] * e0
    num1 = acc[:, _HPH:_D] + h[:, _HPH:_D] * e1
    den0 = _den_col(dp, 0) + e0
    den1 = _den_col(dp, _NC) + e1
    gat = jnp.concatenate([num0 / (den0 + 1e-16), num1 / (den1 + 1e-16)],
                          axis=1) + b1_ref[...][None, :]
    xhop = jnp.where(gat > 0.0, gat, jnp.exp(jnp.minimum(gat, 0.0)) - 1.0)
    xcomb = xhop + xskip_ref[...]
    m = jnp.mean(xcomb, axis=1, keepdims=True)
    v = jnp.mean((xcomb - m) ** 2, axis=1, keepdims=True)
    first = ((xcomb - m) / jnp.sqrt(v + 1e-5) * g1_ref[...][None, :]
             + bb1_ref[...][None, :])
    h2 = jnp.dot(first, w2_ref[...], preferred_element_type=_f32)
    a2s_ref[...] = jnp.sum(h2 * as2_ref[...], axis=1, keepdims=True)
    a2d_ref[...] = jnp.sum(h2 * ad2_ref[...], axis=1, keepdims=True)
    h2_ref[...] = h2
    xfin_ref[...] = (jnp.dot(first, wf_ref[...], preferred_element_type=_f32)
                     + bf_ref[...][None, :])


def _stage_e(part_ref, dp_ref, h2_ref, a2s_ref, a2d_ref, xfin_ref,
             b2_ref, g2_ref, bb2_ref, out_ref):
    acc = part_ref[0:_N, :] + part_ref[_N:2 * _N, :]
    e2 = jnp.exp(_lrelu(a2s_ref[...] + a2d_ref[...]))
    num = acc + h2_ref[...] * e2
    den = _den_col(dp_ref[0], 0) + e2
    x2 = num / (den + 1e-16) + b2_ref[...][None, :]
    y = x2 + xfin_ref[...]
    m = jnp.mean(y, axis=1, keepdims=True)
    v = jnp.mean((y - m) ** 2, axis=1, keepdims=True)
    out_ref[...] = ((y - m) / jnp.sqrt(v + 1e-5) * g2_ref[...][None, :]
                    + bb2_ref[...][None, :])


_stage_a_call = pl.pallas_call(
    _stage_a,
    out_shape=[
        jax.ShapeDtypeStruct((_N, _D), _f32),
        jax.ShapeDtypeStruct((_N, 1), _f32),
        jax.ShapeDtypeStruct((_N, 1), _f32),
        jax.ShapeDtypeStruct((_N, 1), _f32),
        jax.ShapeDtypeStruct((_N, 1), _f32),
        jax.ShapeDtypeStruct((_N, _D), _f32),
    ],
)
_BS = 2000
_stage_c_call = pl.pallas_call(
    _stage_c,
    grid=(_N // _BS,),
    in_specs=[
        pl.BlockSpec((2, _BS, _D), lambda i: (0, i, 0)),
        pl.BlockSpec((1, 2 * _NC, _BS), lambda i: (i, 0, 0)),
        pl.BlockSpec((_BS, _D), lambda i: (i, 0)),
        pl.BlockSpec((_BS, 1), lambda i: (i, 0)),
        pl.BlockSpec((_BS, 1), lambda i: (i, 0)),
        pl.BlockSpec((_BS, 1), lambda i: (i, 0)),
        pl.BlockSpec((_BS, 1), lambda i: (i, 0)),
        pl.BlockSpec((_BS, _D), lambda i: (i, 0)),
        pl.BlockSpec((_D,), lambda i: (0,)),
        pl.BlockSpec((_D,), lambda i: (0,)),
        pl.BlockSpec((_D,), lambda i: (0,)),
        pl.BlockSpec((_D, _D), lambda i: (0, 0)),
        pl.BlockSpec((1, _D), lambda i: (0, 0)),
        pl.BlockSpec((1, _D), lambda i: (0, 0)),
        pl.BlockSpec((_D, _D), lambda i: (0, 0)),
        pl.BlockSpec((_D,), lambda i: (0,)),
    ],
    out_specs=[
        pl.BlockSpec((_BS, _D), lambda i: (i, 0)),
        pl.BlockSpec((_BS, 1), lambda i: (i, 0)),
        pl.BlockSpec((_BS, 1), lambda i: (i, 0)),
        pl.BlockSpec((_BS, _D), lambda i: (i, 0)),
    ],
    out_shape=[
        jax.ShapeDtypeStruct((_N, _D), _f32),
        jax.ShapeDtypeStruct((_N, 1), _f32),
        jax.ShapeDtypeStruct((_N, 1), _f32),
        jax.ShapeDtypeStruct((_N, _D), _f32),
    ],
)
_stage_e_call = pl.pallas_call(
    _stage_e,
    out_shape=jax.ShapeDtypeStruct((_N, _D), _f32),
)


@jax.jit
def _run(x, edge_index, W1, att_src1, att_dst1, bias1, W2, att_src2,
         att_dst2, bias2, W_skip, b_skip, W_final, b_final,
         ln1_g, ln1_b, ln2_g, ln2_b):
    src = edge_index[0]
    dst = edge_index[1]
    h1, as0, as1, ad0, ad1, xskip = _stage_a_call(
        x, W1, att_src1, att_dst1, W_skip, b_skip)
    part1, dp1 = _edge_pass2(h1, src, dst,
                             as0.reshape(_N), ad0.reshape(_N),
                             as1.reshape(_N), ad1.reshape(_N))
    dp1 = jnp.swapaxes(dp1.reshape(2 * _NC, _N // _BS, _BS), 0, 1)
    part1 = part1.reshape(2, _N, _D)
    h2, a2s, a2d, xfin = _stage_c_call(
        part1, dp1, h1, as0, as1, ad0, ad1, xskip, bias1, ln1_g, ln1_b,
        W2, att_src2, att_dst2, W_final, b_final)
    part2, dp2 = _edge_pass1(h2, src, dst, a2s.reshape(_N), a2d.reshape(_N))
    dp2 = dp2.reshape(1, _NC, _N)
    return _stage_e_call(part2, dp2, h2, a2s, a2d, xfin, bias2, ln2_g, ln2_b)


def kernel(x, edge_index, W1, att_src1, att_dst1, bias1, W2, att_src2,
           att_dst2, bias2, W_skip, b_skip, W_final, b_final,
           ln1_g, ln1_b, ln2_g, ln2_b):
    return _run(x, edge_index, W1, att_src1, att_dst1, bias1, W2, att_src2,
                att_dst2, bias2, W_skip, b_skip, W_final, b_final,
                ln1_g, ln1_b, ln2_g, ln2_b)


# speedup vs baseline: 93.8695x; 1.1755x over previous
import jax
import jax.numpy as jnp
from jax import lax
from jax.experimental import pallas as pl
from jax.experimental.pallas import tpu as pltpu
from jax.experimental.pallas import tpu_sc as plsc

_N = 10000
_E = 320000
_D = 128
_HPH = 64
_B = 128
_NC = 2
_NS = 16
_BPC = _E // _NC // _B
_BLK_LO = _BPC // _NS
_BLK_REM = _BPC % _NS
_OPT = _N // 8 // _NS
_OREM = (_N // 8) % _NS
_CH = 104
_G16 = _N // 16
_GPT = _G16 // _NS
_GREM = _G16 % _NS
_WPT = _GPT * 16
_f32 = jnp.float32


def _lrelu(a):
    return jnp.where(a >= 0.0, a, 0.2 * a)


_mesh = plsc.VectorSubcoreMesh(core_axis_name="c", subcore_axis_name="s")


def _make_edge_pass(two_heads):
    nhd = 2 if two_heads else 1
    bufset = (
        [pltpu.VMEM((2, _B), jnp.int32)]
        + [pltpu.VMEM((_B, _D), _f32)]
        + [pltpu.VMEM((_B,), _f32) for _ in range(2 * nhd)]
        + [pltpu.VMEM((_B,), _f32) for _ in range(nhd)]
    )
    scratch = bufset + bufset + [
        pltpu.VMEM((640,), _f32),
    ]
    scratch += [pltpu.VMEM_SHARED((_N,), _f32) for _ in range(nhd)]
    scratch += [
        pltpu.VMEM_SHARED((_N, _D), _f32),
        pltpu.SemaphoreType.DMA,
        pltpu.SemaphoreType.DMA,
        pltpu.SemaphoreType.DMA,
        pltpu.SemaphoreType.DMA,
    ]

    def body(*refs):
        it = iter(refs)
        h_hbm = next(it)
        ei_h = next(it)
        as0_h = next(it)
        ad0_h = next(it)
        as1_h = next(it) if two_heads else None
        ad1_h = next(it) if two_heads else None
        feat_o = next(it)
        den_o = next(it)

        def _take_set():
            s = {}
            s["idx"] = next(it)
            s["rows"] = next(it)
            s["a0s"] = next(it)
            s["a0d"] = next(it)
            s["a1s"] = next(it) if two_heads else None
            s["a1d"] = next(it) if two_heads else None
            s["ex0"] = next(it)
            s["ex1"] = next(it) if two_heads else None
            return s

        bufA = _take_set()
        bufB = _take_set()
        z_v = next(it)
        den0_sh = next(it)
        den1_sh = next(it) if two_heads else None
        acc = next(it)
        semA = next(it)
        semB = next(it)
        semSA = next(it)
        semSB = next(it)
        bufA["gsem"] = semA
        bufB["gsem"] = semB
        bufA["ssem"] = semSA
        bufB["ssem"] = semSB
        rows_v = bufA["rows"]

        cid = lax.axis_index("c")
        sid = lax.axis_index("s")
        zero = jnp.zeros((16,), _f32)

        for i in range(640 // 16):
            z_v[pl.ds(i * 16, 16)] = zero

        def _zrow(r, c_):
            for c in range(_D // 16):
                rows_v[r, pl.ds(c * 16, 16)] = zero
            return c_

        lax.fori_loop(0, _CH, _zrow, 0)

        base_row = pl.multiple_of(8 * (_OPT * sid + jnp.minimum(sid, _OREM)), 8)
        for k in range(_OPT * 8 // _CH):
            pltpu.sync_copy(rows_v.at[pl.ds(0, _CH)],
                            acc.at[pl.ds(base_row + k * _CH, _CH)])

        @pl.when(sid < _OREM)
        def _init_tail():
            pltpu.sync_copy(rows_v.at[pl.ds(0, 8)],
                            acc.at[pl.ds(base_row + _OPT * 8, 8)])

        base_w = pl.multiple_of(16 * (_GPT * sid + jnp.minimum(sid, _GREM)), 16)
        pltpu.sync_copy(z_v.at[pl.ds(0, _WPT)],
                        den0_sh.at[pl.ds(base_w, _WPT)])
        if two_heads:
            pltpu.sync_copy(z_v.at[pl.ds(0, _WPT)],
                            den1_sh.at[pl.ds(base_w, _WPT)])

        @pl.when(sid < _GREM)
        def _init_tail_w():
            pltpu.sync_copy(z_v.at[pl.ds(0, 16)],
                            den0_sh.at[pl.ds(base_w + _WPT, 16)])
            if two_heads:
                pltpu.sync_copy(z_v.at[pl.ds(0, 16)],
                                den1_sh.at[pl.ds(base_w + _WPT, 16)])

        plsc.subcore_barrier()

        blk0 = cid * _BPC + _BLK_LO * sid + jnp.minimum(sid, _BLK_REM)
        last_blk = _E // _B - 1

        def _fire(blk, buf):
            base = blk * _B
            sem = buf["gsem"]
            pltpu.sync_copy(ei_h.at[:, pl.ds(base, _B)], buf["idx"])
            srcr = buf["idx"].at[0]
            dstr = buf["idx"].at[1]
            pltpu.async_copy(as0_h.at[srcr], buf["a0s"], sem)
            pltpu.async_copy(ad0_h.at[dstr], buf["a0d"], sem)
            if two_heads:
                pltpu.async_copy(as1_h.at[srcr], buf["a1s"], sem)
                pltpu.async_copy(ad1_h.at[dstr], buf["a1d"], sem)
            pltpu.async_copy(h_hbm.at[srcr], buf["rows"], sem)

        def _drain(buf):
            sem = buf["gsem"]
            srcr = buf["idx"].at[0]
            dstr = buf["idx"].at[1]
            pltpu.make_async_copy(as0_h.at[srcr], buf["a0s"], sem).wait()
            pltpu.make_async_copy(ad0_h.at[dstr], buf["a0d"], sem).wait()
            if two_heads:
                pltpu.make_async_copy(as1_h.at[srcr], buf["a1s"], sem).wait()
                pltpu.make_async_copy(ad1_h.at[dstr], buf["a1d"], sem).wait()
            pltpu.make_async_copy(h_hbm.at[srcr], buf["rows"], sem).wait()

        def _compute(buf):
            rows = buf["rows"]
            ex0_v = buf["ex0"]
            ex1_v = buf["ex1"]

            def _grp(g, c_):
                off = g * 16
                a0 = buf["a0s"][pl.ds(off, 16)] + buf["a0d"][pl.ds(off, 16)]
                ex0 = jnp.exp(_lrelu(a0))
                ex0_v[pl.ds(off, 16)] = ex0
                if two_heads:
                    a1 = buf["a1s"][pl.ds(off, 16)] + buf["a1d"][pl.ds(off, 16)]
                    ex1 = jnp.exp(_lrelu(a1))
                    ex1_v[pl.ds(off, 16)] = ex1
                for j in range(16):
                    e = off + j
                    idx = jnp.full((16,), e, jnp.int32)
                    v0 = plsc.load_gather(ex0_v, [idx])
                    v1 = plsc.load_gather(ex1_v, [idx]) if two_heads else v0
                    for c in range(4):
                        rows[e, pl.ds(c * 16, 16)] = rows[e, pl.ds(c * 16, 16)] * v0
                    for c in range(4, 8):
                        rows[e, pl.ds(c * 16, 16)] = rows[e, pl.ds(c * 16, 16)] * v1
                return c_

            lax.fori_loop(0, _B // 16, _grp, 0)

        def _scatter(buf):
            sem = buf["ssem"]
            dstr = buf["idx"].at[1]
            pltpu.async_copy(buf["rows"], acc.at[dstr], sem, add=True)
            pltpu.async_copy(buf["ex0"], den0_sh.at[dstr], sem, add=True)
            if two_heads:
                pltpu.async_copy(buf["ex1"], den1_sh.at[dstr], sem, add=True)

        def _drain_scatter(buf):
            sem = buf["ssem"]
            dstr = buf["idx"].at[1]
            pltpu.make_async_copy(buf["rows"], acc.at[dstr], sem).wait()
            pltpu.make_async_copy(buf["ex0"], den0_sh.at[dstr], sem).wait()
            if two_heads:
                pltpu.make_async_copy(buf["ex1"], den1_sh.at[dstr], sem).wait()

        _fire(blk0, bufA)

        def _pair(k, carry):
            bA = blk0 + 2 * k
            _fire(bA + 1, bufB)
            _drain(bufA)
            _compute(bufA)
            _scatter(bufA)
            _drain(bufB)
            _compute(bufB)
            _scatter(bufB)
            _drain_scatter(bufA)
            _fire(jnp.minimum(bA + 2, last_blk), bufA)
            _drain_scatter(bufB)
            return carry

        lax.fori_loop(0, _BLK_LO // 2, _pair, 0)

        _drain(bufA)

        @pl.when(sid < _BLK_REM)
        def _tail_block():
            _compute(bufA)
            _scatter(bufA)
            _drain_scatter(bufA)

        plsc.subcore_barrier()
        for k in range(_OPT * 8 // _CH):
            pltpu.sync_copy(acc.at[pl.ds(base_row + k * _CH, _CH)],
                            feat_o.at[pl.ds(cid * _N + base_row + k * _CH, _CH)])
        pltpu.sync_copy(den0_sh.at[pl.ds(base_w, _WPT)], z_v.at[pl.ds(0, _WPT)])
        pltpu.sync_copy(z_v.at[pl.ds(0, _WPT)],
                        den_o.at[pl.ds(cid * _N + base_w, _WPT)])
        if two_heads:
            pltpu.sync_copy(den1_sh.at[pl.ds(base_w, _WPT)],
                            z_v.at[pl.ds(0, _WPT)])
            pltpu.sync_copy(z_v.at[pl.ds(0, _WPT)],
                            den_o.at[pl.ds((_NC + cid) * _N + base_w, _WPT)])

        @pl.when(sid < _OREM)
        def _drain_tail():
            pltpu.sync_copy(acc.at[pl.ds(base_row + _OPT * 8, 8)],
                            feat_o.at[pl.ds(cid * _N + base_row + _OPT * 8, 8)])

        @pl.when(sid < _GREM)
        def _drain_tail_w():
            pltpu.sync_copy(den0_sh.at[pl.ds(base_w + _WPT, 16)],
                            z_v.at[pl.ds(0, 16)])
            pltpu.sync_copy(z_v.at[pl.ds(0, 16)],
                            den_o.at[pl.ds(cid * _N + base_w + _WPT, 16)])
            if two_heads:
                pltpu.sync_copy(den1_sh.at[pl.ds(base_w + _WPT, 16)],
                                z_v.at[pl.ds(0, 16)])
                pltpu.sync_copy(z_v.at[pl.ds(0, 16)],
                                den_o.at[pl.ds((_NC + cid) * _N + base_w + _WPT, 16)])

    return pl.kernel(
        body,
        out_type=(
            jax.ShapeDtypeStruct((_NC * _N, _D), _f32),
            jax.ShapeDtypeStruct((nhd * _NC * _N,), _f32),
        ),
        mesh=_mesh,
        scratch_types=scratch,
        compiler_params=pltpu.CompilerParams(needs_layout_passes=False),
    )


_edge_pass2 = _make_edge_pass(True)
_edge_pass1 = _make_edge_pass(False)


def _den_col(dp, lo):
    return lax.dot_general(dp[lo:lo + _NC, :], jnp.ones((_NC, 1), _f32),
                           (((0,), (0,)), ((), ())),
                           preferred_element_type=_f32)


def _stage_a(x_ref, w1_ref, as1_ref, ad1_ref, wsk_ref, bsk_ref,
             h_ref, as0_o, as1_o, ad0_o, ad1_o, xskip_ref):
    x = x_ref[...]
    h = jnp.dot(x, w1_ref[...], preferred_element_type=_f32)
    att_s = as1_ref[...]
    att_d = ad1_ref[...]
    h0 = h[:, :_HPH]
    h1 = h[:, _HPH:]
    as0_o[...] = jnp.sum(h0 * att_s[0:1, :], axis=1, keepdims=True)
    as1_o[...] = jnp.sum(h1 * att_s[1:2, :], axis=1, keepdims=True)
    ad0_o[...] = jnp.sum(h0 * att_d[0:1, :], axis=1, keepdims=True)
    ad1_o[...] = jnp.sum(h1 * att_d[1:2, :], axis=1, keepdims=True)
    h_ref[...] = h
    xskip_ref[...] = (jnp.dot(x, wsk_ref[...], preferred_element_type=_f32)
                      + bsk_ref[...][None, :])


def _stage_c(part_ref, dp_ref, h1_ref, as0_ref, as1_ref, ad0_ref, ad1_ref,
             xskip_ref, b1_ref, g1_ref, bb1_ref, w2_ref, as2_ref, ad2_ref,
             wf_ref, bf_ref,
             h2_ref, a2s_ref, a2d_ref, xfin_ref):
    acc = part_ref[0] + part_ref[1]
    dp = dp_ref[0]
    e0 = jnp.exp(_lrelu(as0_ref[...] + ad0_ref[...]))
    e1 = jnp.exp(_lrelu(as1_ref[...] + ad1_ref[...]))
    h = h1_ref[...]
    num0 = acc[:, :_HPH] + h[:, :_HPH] * e0
    num1 = acc[:, _HPH:_D] + h[:, _HPH:_D] * e1
    den0 = _den_col(dp, 0) + e0
    den1 = _den_col(dp, _NC) + e1
    gat = jnp.concatenate([num0 / (den0 + 1e-16), num1 / (den1 + 1e-16)],
                          axis=1) + b1_ref[...][None, :]
    xhop = jnp.where(gat > 0.0, gat, jnp.exp(jnp.minimum(gat, 0.0)) - 1.0)
    xcomb = xhop + xskip_ref[...]
    m = jnp.mean(xcomb, axis=1, keepdims=True)
    v = jnp.mean((xcomb - m) ** 2, axis=1, keepdims=True)
    first = ((xcomb - m) / jnp.sqrt(v + 1e-5) * g1_ref[...][None, :]
             + bb1_ref[...][None, :])
    h2 = jnp.dot(first, w2_ref[...], preferred_element_type=_f32)
    a2s_ref[...] = jnp.sum(h2 * as2_ref[...], axis=1, keepdims=True)
    a2d_ref[...] = jnp.sum(h2 * ad2_ref[...], axis=1, keepdims=True)
    h2_ref[...] = h2
    xfin_ref[...] = (jnp.dot(first, wf_ref[...], preferred_element_type=_f32)
                     + bf_ref[...][None, :])


def _stage_e(part_ref, dp_ref, h2_ref, a2s_ref, a2d_ref, xfin_ref,
             b2_ref, g2_ref, bb2_ref, out_ref):
    acc = part_ref[0:_N, :] + part_ref[_N:2 * _N, :]
    e2 = jnp.exp(_lrelu(a2s_ref[...] + a2d_ref[...]))
    num = acc + h2_ref[...] * e2
    den = _den_col(dp_ref[0], 0) + e2
    x2 = num / (den + 1e-16) + b2_ref[...][None, :]
    y = x2 + xfin_ref[...]
    m = jnp.mean(y, axis=1, keepdims=True)
    v = jnp.mean((y - m) ** 2, axis=1, keepdims=True)
    out_ref[...] = ((y - m) / jnp.sqrt(v + 1e-5) * g2_ref[...][None, :]
                    + bb2_ref[...][None, :])


_stage_a_call = pl.pallas_call(
    _stage_a,
    out_shape=[
        jax.ShapeDtypeStruct((_N, _D), _f32),
        jax.ShapeDtypeStruct((_N, 1), _f32),
        jax.ShapeDtypeStruct((_N, 1), _f32),
        jax.ShapeDtypeStruct((_N, 1), _f32),
        jax.ShapeDtypeStruct((_N, 1), _f32),
        jax.ShapeDtypeStruct((_N, _D), _f32),
    ],
)
_BS = 2000
_stage_c_call = pl.pallas_call(
    _stage_c,
    grid=(_N // _BS,),
    in_specs=[
        pl.BlockSpec((2, _BS, _D), lambda i: (0, i, 0)),
        pl.BlockSpec((1, 2 * _NC, _BS), lambda i: (i, 0, 0)),
        pl.BlockSpec((_BS, _D), lambda i: (i, 0)),
        pl.BlockSpec((_BS, 1), lambda i: (i, 0)),
        pl.BlockSpec((_BS, 1), lambda i: (i, 0)),
        pl.BlockSpec((_BS, 1), lambda i: (i, 0)),
        pl.BlockSpec((_BS, 1), lambda i: (i, 0)),
        pl.BlockSpec((_BS, _D), lambda i: (i, 0)),
        pl.BlockSpec((_D,), lambda i: (0,)),
        pl.BlockSpec((_D,), lambda i: (0,)),
        pl.BlockSpec((_D,), lambda i: (0,)),
        pl.BlockSpec((_D, _D), lambda i: (0, 0)),
        pl.BlockSpec((1, _D), lambda i: (0, 0)),
        pl.BlockSpec((1, _D), lambda i: (0, 0)),
        pl.BlockSpec((_D, _D), lambda i: (0, 0)),
        pl.BlockSpec((_D,), lambda i: (0,)),
    ],
    out_specs=[
        pl.BlockSpec((_BS, _D), lambda i: (i, 0)),
        pl.BlockSpec((_BS, 1), lambda i: (i, 0)),
        pl.BlockSpec((_BS, 1), lambda i: (i, 0)),
        pl.BlockSpec((_BS, _D), lambda i: (i, 0)),
    ],
    out_shape=[
        jax.ShapeDtypeStruct((_N, _D), _f32),
        jax.ShapeDtypeStruct((_N, 1), _f32),
        jax.ShapeDtypeStruct((_N, 1), _f32),
        jax.ShapeDtypeStruct((_N, _D), _f32),
    ],
)
_stage_e_call = pl.pallas_call(
    _stage_e,
    out_shape=jax.ShapeDtypeStruct((_N, _D), _f32),
)


@jax.jit
def _run(x, edge_index, W1, att_src1, att_dst1, bias1, W2, att_src2,
         att_dst2, bias2, W_skip, b_skip, W_final, b_final,
         ln1_g, ln1_b, ln2_g, ln2_b):
    h1, as0, as1, ad0, ad1, xskip = _stage_a_call(
        x, W1, att_src1, att_dst1, W_skip, b_skip)
    part1, dp1 = _edge_pass2(h1, edge_index,
                             as0.reshape(_N), ad0.reshape(_N),
                             as1.reshape(_N), ad1.reshape(_N))
    dp1 = jnp.swapaxes(dp1.reshape(2 * _NC, _N // _BS, _BS), 0, 1)
    part1 = part1.reshape(2, _N, _D)
    h2, a2s, a2d, xfin = _stage_c_call(
        part1, dp1, h1, as0, as1, ad0, ad1, xskip, bias1, ln1_g, ln1_b,
        W2, att_src2, att_dst2, W_final, b_final)
    part2, dp2 = _edge_pass1(h2, edge_index, a2s.reshape(_N), a2d.reshape(_N))
    dp2 = dp2.reshape(1, _NC, _N)
    return _stage_e_call(part2, dp2, h2, a2s, a2d, xfin, bias2, ln2_g, ln2_b)


def kernel(x, edge_index, W1, att_src1, att_dst1, bias1, W2, att_src2,
           att_dst2, bias2, W_skip, b_skip, W_final, b_final,
           ln1_g, ln1_b, ln2_g, ln2_b):
    return _run(x, edge_index, W1, att_src1, att_dst1, bias1, W2, att_src2,
                att_dst2, bias2, W_skip, b_skip, W_final, b_final,
                ln1_g, ln1_b, ln2_g, ln2_b)


# speedup vs baseline: 97.5962x; 1.0397x over previous
import jax
import jax.numpy as jnp
from jax import lax
from jax.experimental import pallas as pl
from jax.experimental.pallas import tpu as pltpu
from jax.experimental.pallas import tpu_sc as plsc

_N = 10000
_E = 320000
_D = 128
_HPH = 64
_B = 128
_NC = 2
_NS = 16
_BPC = _E // _NC // _B
_BLK_LO = _BPC // _NS
_BLK_REM = _BPC % _NS
_OPT = _N // 8 // _NS
_OREM = (_N // 8) % _NS
_CH = 104
_NBLK = _E // _B
_DPAD = _NBLK + 4
_DSZ = 88
_G16 = _N // 16
_GPT = _G16 // _NS
_GREM = _G16 % _NS
_WPT = _GPT * 16
_f32 = jnp.float32


def _lrelu(a):
    return jnp.where(a >= 0.0, a, 0.2 * a)


_mesh = plsc.VectorSubcoreMesh(core_axis_name="c", subcore_axis_name="s")


def _make_edge_pass(two_heads):
    nhd = 2 if two_heads else 1
    bufset = (
        [pltpu.VMEM((_B,), jnp.int32)]
        + [pltpu.VMEM((_B, _D), _f32)]
        + [pltpu.VMEM((_B,), _f32) for _ in range(2 * nhd)]
        + [pltpu.VMEM((_B,), _f32) for _ in range(nhd)]
    )
    scratch = bufset + bufset + [
        pltpu.VMEM((_DSZ, _B), jnp.int32),
        pltpu.VMEM((640,), _f32),
    ]
    scratch += [pltpu.VMEM_SHARED((_N,), _f32) for _ in range(nhd)]
    scratch += [
        pltpu.VMEM_SHARED((_N, _D), _f32),
        pltpu.SemaphoreType.DMA,
        pltpu.SemaphoreType.DMA,
        pltpu.SemaphoreType.DMA,
        pltpu.SemaphoreType.DMA,
        pltpu.SemaphoreType.DMA,
        pltpu.SemaphoreType.DMA,
    ]

    def body(*refs):
        it = iter(refs)
        h_hbm = next(it)
        src_h = next(it)
        dst2_h = next(it)
        as0_h = next(it)
        ad0_h = next(it)
        as1_h = next(it) if two_heads else None
        ad1_h = next(it) if two_heads else None
        feat_o = next(it)
        den_o = next(it)

        def _take_set():
            s = {}
            s["src"] = next(it)
            s["rows"] = next(it)
            s["a0s"] = next(it)
            s["a0d"] = next(it)
            s["a1s"] = next(it) if two_heads else None
            s["a1d"] = next(it) if two_heads else None
            s["ex0"] = next(it)
            s["ex1"] = next(it) if two_heads else None
            return s

        bufA = _take_set()
        bufB = _take_set()
        dstall_v = next(it)
        z_v = next(it)
        den0_sh = next(it)
        den1_sh = next(it) if two_heads else None
        acc = next(it)
        bufA["gsem"] = next(it)
        bufB["gsem"] = next(it)
        bufA["ssem"] = next(it)
        bufB["ssem"] = next(it)
        bufA["isem"] = next(it)
        bufB["isem"] = next(it)
        rows_v = bufA["rows"]

        cid = lax.axis_index("c")
        sid = lax.axis_index("s")
        zero = jnp.zeros((16,), _f32)

        for i in range(640 // 16):
            z_v[pl.ds(i * 16, 16)] = zero

        def _zrow(r, c_):
            for c in range(_D // 16):
                rows_v[r, pl.ds(c * 16, 16)] = zero
            return c_

        lax.fori_loop(0, _CH, _zrow, 0)

        base_row = pl.multiple_of(8 * (_OPT * sid + jnp.minimum(sid, _OREM)), 8)
        for k in range(_OPT * 8 // _CH):
            pltpu.sync_copy(rows_v.at[pl.ds(0, _CH)],
                            acc.at[pl.ds(base_row + k * _CH, _CH)])

        @pl.when(sid < _OREM)
        def _init_tail():
            pltpu.sync_copy(rows_v.at[pl.ds(0, 8)],
                            acc.at[pl.ds(base_row + _OPT * 8, 8)])

        base_w = pl.multiple_of(16 * (_GPT * sid + jnp.minimum(sid, _GREM)), 16)
        pltpu.sync_copy(z_v.at[pl.ds(0, _WPT)],
                        den0_sh.at[pl.ds(base_w, _WPT)])
        if two_heads:
            pltpu.sync_copy(z_v.at[pl.ds(0, _WPT)],
                            den1_sh.at[pl.ds(base_w, _WPT)])

        @pl.when(sid < _GREM)
        def _init_tail_w():
            pltpu.sync_copy(z_v.at[pl.ds(0, 16)],
                            den0_sh.at[pl.ds(base_w + _WPT, 16)])
            if two_heads:
                pltpu.sync_copy(z_v.at[pl.ds(0, 16)],
                                den1_sh.at[pl.ds(base_w + _WPT, 16)])

        plsc.subcore_barrier()

        blk0 = cid * _BPC + _BLK_LO * sid + jnp.minimum(sid, _BLK_REM)
        last_blk = _E // _B - 1

        start8 = pl.multiple_of(
            jnp.minimum(8 * (blk0 // 8), _DPAD - _DSZ), 8)
        d0 = blk0 - start8
        pltpu.sync_copy(dst2_h.at[pl.ds(start8, _DSZ)], dstall_v)

        def _dstr(loc):
            return dstall_v.at[d0 + loc]

        def _fire_src(blk, buf):
            pltpu.async_copy(src_h.at[pl.ds(blk * _B, _B)], buf["src"],
                             buf["isem"])

        def _wait_src(buf):
            pltpu.make_async_copy(src_h.at[pl.ds(0, _B)], buf["src"],
                                  buf["isem"]).wait()

        def _fire(buf, loc):
            sem = buf["gsem"]
            srcr = buf["src"]
            dstr = _dstr(loc)
            pltpu.async_copy(as0_h.at[srcr], buf["a0s"], sem)
            pltpu.async_copy(ad0_h.at[dstr], buf["a0d"], sem)
            if two_heads:
                pltpu.async_copy(as1_h.at[srcr], buf["a1s"], sem)
                pltpu.async_copy(ad1_h.at[dstr], buf["a1d"], sem)
            pltpu.async_copy(h_hbm.at[srcr], buf["rows"], sem)

        def _drain(buf, loc):
            sem = buf["gsem"]
            srcr = buf["src"]
            dstr = _dstr(loc)
            pltpu.make_async_copy(as0_h.at[srcr], buf["a0s"], sem).wait()
            pltpu.make_async_copy(ad0_h.at[dstr], buf["a0d"], sem).wait()
            if two_heads:
                pltpu.make_async_copy(as1_h.at[srcr], buf["a1s"], sem).wait()
                pltpu.make_async_copy(ad1_h.at[dstr], buf["a1d"], sem).wait()
            pltpu.make_async_copy(h_hbm.at[srcr], buf["rows"], sem).wait()

        def _compute(buf):
            rows = buf["rows"]
            ex0_v = buf["ex0"]
            ex1_v = buf["ex1"]

            def _grp(g, c_):
                off = g * 16
                a0 = buf["a0s"][pl.ds(off, 16)] + buf["a0d"][pl.ds(off, 16)]
                ex0 = jnp.exp(_lrelu(a0))
                ex0_v[pl.ds(off, 16)] = ex0
                if two_heads:
                    a1 = buf["a1s"][pl.ds(off, 16)] + buf["a1d"][pl.ds(off, 16)]
                    ex1 = jnp.exp(_lrelu(a1))
                    ex1_v[pl.ds(off, 16)] = ex1
                for j in range(16):
                    e = off + j
                    idx = jnp.full((16,), e, jnp.int32)
                    v0 = plsc.load_gather(ex0_v, [idx])
                    v1 = plsc.load_gather(ex1_v, [idx]) if two_heads else v0
                    for c in range(4):
                        rows[e, pl.ds(c * 16, 16)] = rows[e, pl.ds(c * 16, 16)] * v0
                    for c in range(4, 8):
                        rows[e, pl.ds(c * 16, 16)] = rows[e, pl.ds(c * 16, 16)] * v1
                return c_

            lax.fori_loop(0, _B // 16, _grp, 0)

        def _scatter(buf, loc):
            sem = buf["ssem"]
            dstr = _dstr(loc)
            pltpu.async_copy(buf["rows"], acc.at[dstr], sem, add=True)
            pltpu.async_copy(buf["ex0"], den0_sh.at[dstr], sem, add=True)
            if two_heads:
                pltpu.async_copy(buf["ex1"], den1_sh.at[dstr], sem, add=True)

        def _drain_scatter(buf, loc):
            sem = buf["ssem"]
            dstr = _dstr(loc)
            pltpu.make_async_copy(buf["rows"], acc.at[dstr], sem).wait()
            pltpu.make_async_copy(buf["ex0"], den0_sh.at[dstr], sem).wait()
            if two_heads:
                pltpu.make_async_copy(buf["ex1"], den1_sh.at[dstr], sem).wait()

        pltpu.sync_copy(src_h.at[pl.ds(blk0 * _B, _B)], bufA["src"])
        _fire(bufA, 0)
        _fire_src(blk0 + 1, bufB)

        def _pair(k, carry):
            bA = blk0 + 2 * k
            lA = 2 * k
            _wait_src(bufB)
            _fire(bufB, lA + 1)
            _drain(bufA, lA)
            _fire_src(jnp.minimum(bA + 2, last_blk), bufA)
            _compute(bufA)
            _scatter(bufA, lA)
            _drain(bufB, lA + 1)
            _fire_src(jnp.minimum(bA + 3, last_blk), bufB)
            _compute(bufB)
            _scatter(bufB, lA + 1)
            _drain_scatter(bufA, lA)
            _wait_src(bufA)
            _fire(bufA, jnp.minimum(lA + 2, _BLK_LO))
            _drain_scatter(bufB, lA + 1)
            return carry

        lax.fori_loop(0, _BLK_LO // 2, _pair, 0)

        _drain(bufA, _BLK_LO)
        _wait_src(bufB)

        @pl.when(sid < _BLK_REM)
        def _tail_block():
            _compute(bufA)
            _scatter(bufA, _BLK_LO)
            _drain_scatter(bufA, _BLK_LO)

        plsc.subcore_barrier()
        for k in range(_OPT * 8 // _CH):
            pltpu.sync_copy(acc.at[pl.ds(base_row + k * _CH, _CH)],
                            feat_o.at[pl.ds(cid * _N + base_row + k * _CH, _CH)])
        pltpu.sync_copy(den0_sh.at[pl.ds(base_w, _WPT)], z_v.at[pl.ds(0, _WPT)])
        pltpu.sync_copy(z_v.at[pl.ds(0, _WPT)],
                        den_o.at[pl.ds(cid * _N + base_w, _WPT)])
        if two_heads:
            pltpu.sync_copy(den1_sh.at[pl.ds(base_w, _WPT)],
                            z_v.at[pl.ds(0, _WPT)])
            pltpu.sync_copy(z_v.at[pl.ds(0, _WPT)],
                            den_o.at[pl.ds((_NC + cid) * _N + base_w, _WPT)])

        @pl.when(sid < _OREM)
        def _drain_tail():
            pltpu.sync_copy(acc.at[pl.ds(base_row + _OPT * 8, 8)],
                            feat_o.at[pl.ds(cid * _N + base_row + _OPT * 8, 8)])

        @pl.when(sid < _GREM)
        def _drain_tail_w():
            pltpu.sync_copy(den0_sh.at[pl.ds(base_w + _WPT, 16)],
                            z_v.at[pl.ds(0, 16)])
            pltpu.sync_copy(z_v.at[pl.ds(0, 16)],
                            den_o.at[pl.ds(cid * _N + base_w + _WPT, 16)])
            if two_heads:
                pltpu.sync_copy(den1_sh.at[pl.ds(base_w + _WPT, 16)],
                                z_v.at[pl.ds(0, 16)])
                pltpu.sync_copy(z_v.at[pl.ds(0, 16)],
                                den_o.at[pl.ds((_NC + cid) * _N + base_w + _WPT, 16)])

    return pl.kernel(
        body,
        out_type=(
            jax.ShapeDtypeStruct((_NC * _N, _D), _f32),
            jax.ShapeDtypeStruct((nhd * _NC * _N,), _f32),
        ),
        mesh=_mesh,
        scratch_types=scratch,
        compiler_params=pltpu.CompilerParams(needs_layout_passes=False),
    )


_edge_pass2 = _make_edge_pass(True)
_edge_pass1 = _make_edge_pass(False)


def _den_col(dp, lo):
    return lax.dot_general(dp[lo:lo + _NC, :], jnp.ones((_NC, 1), _f32),
                           (((0,), (0,)), ((), ())),
                           preferred_element_type=_f32)


def _stage_a(x_ref, w1_ref, as1_ref, ad1_ref, wsk_ref, bsk_ref,
             h_ref, as0_o, as1_o, ad0_o, ad1_o, xskip_ref):
    x = x_ref[...]
    h = jnp.dot(x, w1_ref[...], preferred_element_type=_f32)
    att_s = as1_ref[...]
    att_d = ad1_ref[...]
    h0 = h[:, :_HPH]
    h1 = h[:, _HPH:]
    as0_o[...] = jnp.sum(h0 * att_s[0:1, :], axis=1, keepdims=True)
    as1_o[...] = jnp.sum(h1 * att_s[1:2, :], axis=1, keepdims=True)
    ad0_o[...] = jnp.sum(h0 * att_d[0:1, :], axis=1, keepdims=True)
    ad1_o[...] = jnp.sum(h1 * att_d[1:2, :], axis=1, keepdims=True)
    h_ref[...] = h
    xskip_ref[...] = (jnp.dot(x, wsk_ref[...], preferred_element_type=_f32)
                      + bsk_ref[...][None, :])


def _stage_c(part_ref, dp_ref, h1_ref, as0_ref, as1_ref, ad0_ref, ad1_ref,
             xskip_ref, b1_ref, g1_ref, bb1_ref, w2_ref, as2_ref, ad2_ref,
             wf_ref, bf_ref,
             h2_ref, a2s_ref, a2d_ref, xfin_ref):
    acc = part_ref[0] + part_ref[1]
    dp = dp_ref[0]
    e0 = jnp.exp(_lrelu(as0_ref[...] + ad0_ref[...]))
    e1 = jnp.exp(_lrelu(as1_ref[...] + ad1_ref[...]))
    h = h1_ref[...]
    num0 = acc[:, :_HPH] + h[:, :_HPH] * e0
    num1 = acc[:, _HPH:_D] + h[:, _HPH:_D] * e1
    den0 = _den_col(dp, 0) + e0
    den1 = _den_col(dp, _NC) + e1
    gat = jnp.concatenate([num0 / (den0 + 1e-16), num1 / (den1 + 1e-16)],
                          axis=1) + b1_ref[...][None, :]
    xhop = jnp.where(gat > 0.0, gat, jnp.exp(jnp.minimum(gat, 0.0)) - 1.0)
    xcomb = xhop + xskip_ref[...]
    m = jnp.mean(xcomb, axis=1, keepdims=True)
    v = jnp.mean((xcomb - m) ** 2, axis=1, keepdims=True)
    first = ((xcomb - m) / jnp.sqrt(v + 1e-5) * g1_ref[...][None, :]
             + bb1_ref[...][None, :])
    h2 = jnp.dot(first, w2_ref[...], preferred_element_type=_f32)
    a2s_ref[...] = jnp.sum(h2 * as2_ref[...], axis=1, keepdims=True)
    a2d_ref[...] = jnp.sum(h2 * ad2_ref[...], axis=1, keepdims=True)
    h2_ref[...] = h2
    xfin_ref[...] = (jnp.dot(first, wf_ref[...], preferred_element_type=_f32)
                     + bf_ref[...][None, :])


def _stage_e(part_ref, dp_ref, h2_ref, a2s_ref, a2d_ref, xfin_ref,
             b2_ref, g2_ref, bb2_ref, out_ref):
    acc = part_ref[0:_N, :] + part_ref[_N:2 * _N, :]
    e2 = jnp.exp(_lrelu(a2s_ref[...] + a2d_ref[...]))
    num = acc + h2_ref[...] * e2
    den = _den_col(dp_ref[0], 0) + e2
    x2 = num / (den + 1e-16) + b2_ref[...][None, :]
    y = x2 + xfin_ref[...]
    m = jnp.mean(y, axis=1, keepdims=True)
    v = jnp.mean((y - m) ** 2, axis=1, keepdims=True)
    out_ref[...] = ((y - m) / jnp.sqrt(v + 1e-5) * g2_ref[...][None, :]
                    + bb2_ref[...][None, :])


_stage_a_call = pl.pallas_call(
    _stage_a,
    out_shape=[
        jax.ShapeDtypeStruct((_N, _D), _f32),
        jax.ShapeDtypeStruct((_N, 1), _f32),
        jax.ShapeDtypeStruct((_N, 1), _f32),
        jax.ShapeDtypeStruct((_N, 1), _f32),
        jax.ShapeDtypeStruct((_N, 1), _f32),
        jax.ShapeDtypeStruct((_N, _D), _f32),
    ],
)
_BS = 2000
_stage_c_call = pl.pallas_call(
    _stage_c,
    grid=(_N // _BS,),
    in_specs=[
        pl.BlockSpec((2, _BS, _D), lambda i: (0, i, 0)),
        pl.BlockSpec((1, 2 * _NC, _BS), lambda i: (i, 0, 0)),
        pl.BlockSpec((_BS, _D), lambda i: (i, 0)),
        pl.BlockSpec((_BS, 1), lambda i: (i, 0)),
        pl.BlockSpec((_BS, 1), lambda i: (i, 0)),
        pl.BlockSpec((_BS, 1), lambda i: (i, 0)),
        pl.BlockSpec((_BS, 1), lambda i: (i, 0)),
        pl.BlockSpec((_BS, _D), lambda i: (i, 0)),
        pl.BlockSpec((_D,), lambda i: (0,)),
        pl.BlockSpec((_D,), lambda i: (0,)),
        pl.BlockSpec((_D,), lambda i: (0,)),
        pl.BlockSpec((_D, _D), lambda i: (0, 0)),
        pl.BlockSpec((1, _D), lambda i: (0, 0)),
        pl.BlockSpec((1, _D), lambda i: (0, 0)),
        pl.BlockSpec((_D, _D), lambda i: (0, 0)),
        pl.BlockSpec((_D,), lambda i: (0,)),
    ],
    out_specs=[
        pl.BlockSpec((_BS, _D), lambda i: (i, 0)),
        pl.BlockSpec((_BS, 1), lambda i: (i, 0)),
        pl.BlockSpec((_BS, 1), lambda i: (i, 0)),
        pl.BlockSpec((_BS, _D), lambda i: (i, 0)),
    ],
    out_shape=[
        jax.ShapeDtypeStruct((_N, _D), _f32),
        jax.ShapeDtypeStruct((_N, 1), _f32),
        jax.ShapeDtypeStruct((_N, 1), _f32),
        jax.ShapeDtypeStruct((_N, _D), _f32),
    ],
)
_stage_e_call = pl.pallas_call(
    _stage_e,
    out_shape=jax.ShapeDtypeStruct((_N, _D), _f32),
)


@jax.jit
def _run(x, edge_index, W1, att_src1, att_dst1, bias1, W2, att_src2,
         att_dst2, bias2, W_skip, b_skip, W_final, b_final,
         ln1_g, ln1_b, ln2_g, ln2_b):
    src = edge_index[0]
    dst2 = jnp.concatenate(
        [edge_index[1].reshape(_NBLK, _B),
         jnp.zeros((_DPAD - _NBLK, _B), jnp.int32)], axis=0)
    h1, as0, as1, ad0, ad1, xskip = _stage_a_call(
        x, W1, att_src1, att_dst1, W_skip, b_skip)
    part1, dp1 = _edge_pass2(h1, src, dst2,
                             as0.reshape(_N), ad0.reshape(_N),
                             as1.reshape(_N), ad1.reshape(_N))
    dp1 = jnp.swapaxes(dp1.reshape(2 * _NC, _N // _BS, _BS), 0, 1)
    part1 = part1.reshape(2, _N, _D)
    h2, a2s, a2d, xfin = _stage_c_call(
        part1, dp1, h1, as0, as1, ad0, ad1, xskip, bias1, ln1_g, ln1_b,
        W2, att_src2, att_dst2, W_final, b_final)
    part2, dp2 = _edge_pass1(h2, src, dst2, a2s.reshape(_N), a2d.reshape(_N))
    dp2 = dp2.reshape(1, _NC, _N)
    return _stage_e_call(part2, dp2, h2, a2s, a2d, xfin, bias2, ln2_g, ln2_b)


def kernel(x, edge_index, W1, att_src1, att_dst1, bias1, W2, att_src2,
           att_dst2, bias2, W_skip, b_skip, W_final, b_final,
           ln1_g, ln1_b, ln2_g, ln2_b):
    return _run(x, edge_index, W1, att_src1, att_dst1, bias1, W2, att_src2,
                att_dst2, bias2, W_skip, b_skip, W_final, b_final,
                ln1_g, ln1_b, ln2_g, ln2_b)


# speedup vs baseline: 109.3117x; 1.1200x over previous
import jax
import jax.numpy as jnp
from jax import lax
from jax.experimental import pallas as pl
from jax.experimental.pallas import tpu as pltpu
from jax.experimental.pallas import tpu_sc as plsc

_N = 10000
_E = 320000
_D = 128
_HPH = 64
_B = 128
_NC = 2
_NS = 16
_BPC = _E // _NC // _B
_BLK_LO = _BPC // _NS
_BLK_REM = _BPC % _NS
_OPT = _N // 8 // _NS
_OREM = (_N // 8) % _NS
_CH = 104
_NBLK = _E // _B
_DPAD = _NBLK + 4
_DSZ = 88
_G16 = _N // 16
_GPT = _G16 // _NS
_GREM = _G16 % _NS
_WPT = _GPT * 16
_f32 = jnp.float32


def _lrelu(a):
    return jnp.where(a >= 0.0, a, 0.2 * a)


_mesh = plsc.VectorSubcoreMesh(core_axis_name="c", subcore_axis_name="s")


def _make_edge_pass(two_heads):
    nhd = 2 if two_heads else 1
    bufset = (
        [pltpu.VMEM((_B,), jnp.int32)]
        + [pltpu.VMEM((_B, _D), _f32)]
        + [pltpu.VMEM((_B,), _f32) for _ in range(2 * nhd)]
        + [pltpu.VMEM((_B,), _f32) for _ in range(nhd)]
    )
    scratch = bufset + bufset + [
        pltpu.VMEM((_DSZ, _B), jnp.int32),
        pltpu.VMEM((640,), _f32),
    ]
    scratch += [pltpu.VMEM_SHARED((_N,), _f32) for _ in range(nhd)]
    scratch += [
        pltpu.VMEM_SHARED((_N, _D), _f32),
        pltpu.SemaphoreType.DMA,
        pltpu.SemaphoreType.DMA,
        pltpu.SemaphoreType.DMA,
        pltpu.SemaphoreType.DMA,
        pltpu.SemaphoreType.DMA,
        pltpu.SemaphoreType.DMA,
    ]

    def body(*refs):
        it = iter(refs)
        h_hbm = next(it)
        src_h = next(it)
        dst2_h = next(it)
        as0_h = next(it)
        ad0_h = next(it)
        as1_h = next(it) if two_heads else None
        ad1_h = next(it) if two_heads else None
        feat_o = next(it)
        den_o = next(it)

        def _take_set():
            s = {}
            s["src"] = next(it)
            s["rows"] = next(it)
            s["a0s"] = next(it)
            s["a0d"] = next(it)
            s["a1s"] = next(it) if two_heads else None
            s["a1d"] = next(it) if two_heads else None
            s["ex0"] = next(it)
            s["ex1"] = next(it) if two_heads else None
            return s

        bufA = _take_set()
        bufB = _take_set()
        dstall_v = next(it)
        z_v = next(it)
        den0_sh = next(it)
        den1_sh = next(it) if two_heads else None
        acc = next(it)
        bufA["gsem"] = next(it)
        bufB["gsem"] = next(it)
        bufA["ssem"] = next(it)
        bufB["ssem"] = next(it)
        bufA["isem"] = next(it)
        bufB["isem"] = next(it)
        rows_v = bufA["rows"]

        cid = lax.axis_index("c")
        sid = lax.axis_index("s")
        zero = jnp.zeros((16,), _f32)

        for i in range(640 // 16):
            z_v[pl.ds(i * 16, 16)] = zero

        def _zrow(r, c_):
            for c in range(_D // 16):
                rows_v[r, pl.ds(c * 16, 16)] = zero
            return c_

        lax.fori_loop(0, _CH, _zrow, 0)

        base_row = pl.multiple_of(8 * (_OPT * sid + jnp.minimum(sid, _OREM)), 8)
        for k in range(_OPT * 8 // _CH):
            pltpu.sync_copy(rows_v.at[pl.ds(0, _CH)],
                            acc.at[pl.ds(base_row + k * _CH, _CH)])

        @pl.when(sid < _OREM)
        def _init_tail():
            pltpu.sync_copy(rows_v.at[pl.ds(0, 8)],
                            acc.at[pl.ds(base_row + _OPT * 8, 8)])

        base_w = pl.multiple_of(16 * (_GPT * sid + jnp.minimum(sid, _GREM)), 16)
        pltpu.sync_copy(z_v.at[pl.ds(0, _WPT)],
                        den0_sh.at[pl.ds(base_w, _WPT)])
        if two_heads:
            pltpu.sync_copy(z_v.at[pl.ds(0, _WPT)],
                            den1_sh.at[pl.ds(base_w, _WPT)])

        @pl.when(sid < _GREM)
        def _init_tail_w():
            pltpu.sync_copy(z_v.at[pl.ds(0, 16)],
                            den0_sh.at[pl.ds(base_w + _WPT, 16)])
            if two_heads:
                pltpu.sync_copy(z_v.at[pl.ds(0, 16)],
                                den1_sh.at[pl.ds(base_w + _WPT, 16)])

        plsc.subcore_barrier()

        blk0 = cid * _BPC + _BLK_LO * sid + jnp.minimum(sid, _BLK_REM)
        last_blk = _E // _B - 1

        start8 = pl.multiple_of(
            jnp.minimum(8 * (blk0 // 8), _DPAD - _DSZ), 8)
        d0 = blk0 - start8
        pltpu.sync_copy(dst2_h.at[pl.ds(start8, _DSZ)], dstall_v)

        def _dstr(loc):
            return dstall_v.at[d0 + loc]

        def _fire_src(blk, buf):
            pltpu.async_copy(src_h.at[pl.ds(blk * _B, _B)], buf["src"],
                             buf["isem"])

        def _wait_src(buf):
            pltpu.make_async_copy(src_h.at[pl.ds(0, _B)], buf["src"],
                                  buf["isem"]).wait()

        def _fire(buf, loc):
            sem = buf["gsem"]
            srcr = buf["src"]
            dstr = _dstr(loc)
            pltpu.async_copy(as0_h.at[srcr], buf["a0s"], sem)
            pltpu.async_copy(ad0_h.at[dstr], buf["a0d"], sem)
            if two_heads:
                pltpu.async_copy(as1_h.at[srcr], buf["a1s"], sem)
                pltpu.async_copy(ad1_h.at[dstr], buf["a1d"], sem)
            pltpu.async_copy(h_hbm.at[srcr], buf["rows"], sem)

        def _drain(buf, loc):
            sem = buf["gsem"]
            srcr = buf["src"]
            dstr = _dstr(loc)
            pltpu.make_async_copy(as0_h.at[srcr], buf["a0s"], sem).wait()
            pltpu.make_async_copy(ad0_h.at[dstr], buf["a0d"], sem).wait()
            if two_heads:
                pltpu.make_async_copy(as1_h.at[srcr], buf["a1s"], sem).wait()
                pltpu.make_async_copy(ad1_h.at[dstr], buf["a1d"], sem).wait()
            pltpu.make_async_copy(h_hbm.at[srcr], buf["rows"], sem).wait()

        def _compute(buf):
            rows = buf["rows"]
            ex0_v = buf["ex0"]
            ex1_v = buf["ex1"]

            def _grp(g, c_):
                off = g * 16
                a0 = buf["a0s"][pl.ds(off, 16)] + buf["a0d"][pl.ds(off, 16)]
                ex0 = jnp.exp(_lrelu(a0))
                ex0_v[pl.ds(off, 16)] = ex0
                if two_heads:
                    a1 = buf["a1s"][pl.ds(off, 16)] + buf["a1d"][pl.ds(off, 16)]
                    ex1 = jnp.exp(_lrelu(a1))
                    ex1_v[pl.ds(off, 16)] = ex1
                else:
                    ex1 = ex0
                for j in range(16):
                    e = off + j
                    idx = jnp.full((16,), j, jnp.int32)
                    v0 = ex0.at[idx].get(mode="promise_in_bounds")
                    v1 = (ex1.at[idx].get(mode="promise_in_bounds")
                          if two_heads else v0)
                    for c in range(4):
                        rows[e, pl.ds(c * 16, 16)] = rows[e, pl.ds(c * 16, 16)] * v0
                    for c in range(4, 8):
                        rows[e, pl.ds(c * 16, 16)] = rows[e, pl.ds(c * 16, 16)] * v1
                return c_

            lax.fori_loop(0, _B // 16, _grp, 0)

        def _scatter(buf, loc):
            sem = buf["ssem"]
            dstr = _dstr(loc)
            pltpu.async_copy(buf["rows"], acc.at[dstr], sem, add=True)
            pltpu.async_copy(buf["ex0"], den0_sh.at[dstr], sem, add=True)
            if two_heads:
                pltpu.async_copy(buf["ex1"], den1_sh.at[dstr], sem, add=True)

        def _drain_scatter(buf, loc):
            sem = buf["ssem"]
            dstr = _dstr(loc)
            pltpu.make_async_copy(buf["rows"], acc.at[dstr], sem).wait()
            pltpu.make_async_copy(buf["ex0"], den0_sh.at[dstr], sem).wait()
            if two_heads:
                pltpu.make_async_copy(buf["ex1"], den1_sh.at[dstr], sem).wait()

        pltpu.sync_copy(src_h.at[pl.ds(blk0 * _B, _B)], bufA["src"])
        _fire(bufA, 0)
        _fire_src(blk0 + 1, bufB)

        def _pair(k, carry):
            bA = blk0 + 2 * k
            lA = 2 * k
            _wait_src(bufB)
            _fire(bufB, lA + 1)
            _drain(bufA, lA)
            _fire_src(jnp.minimum(bA + 2, last_blk), bufA)
            _compute(bufA)
            _scatter(bufA, lA)
            _drain(bufB, lA + 1)
            _fire_src(jnp.minimum(bA + 3, last_blk), bufB)
            _compute(bufB)
            _scatter(bufB, lA + 1)
            _drain_scatter(bufA, lA)
            _wait_src(bufA)
            _fire(bufA, jnp.minimum(lA + 2, _BLK_LO))
            _drain_scatter(bufB, lA + 1)
            return carry

        lax.fori_loop(0, _BLK_LO // 2, _pair, 0)

        _drain(bufA, _BLK_LO)
        _wait_src(bufB)

        @pl.when(sid < _BLK_REM)
        def _tail_block():
            _compute(bufA)
            _scatter(bufA, _BLK_LO)
            _drain_scatter(bufA, _BLK_LO)

        plsc.subcore_barrier()
        for k in range(_OPT * 8 // _CH):
            pltpu.sync_copy(acc.at[pl.ds(base_row + k * _CH, _CH)],
                            feat_o.at[pl.ds(cid * _N + base_row + k * _CH, _CH)])
        pltpu.sync_copy(den0_sh.at[pl.ds(base_w, _WPT)], z_v.at[pl.ds(0, _WPT)])
        pltpu.sync_copy(z_v.at[pl.ds(0, _WPT)],
                        den_o.at[pl.ds(cid * _N + base_w, _WPT)])
        if two_heads:
            pltpu.sync_copy(den1_sh.at[pl.ds(base_w, _WPT)],
                            z_v.at[pl.ds(0, _WPT)])
            pltpu.sync_copy(z_v.at[pl.ds(0, _WPT)],
                            den_o.at[pl.ds((_NC + cid) * _N + base_w, _WPT)])

        @pl.when(sid < _OREM)
        def _drain_tail():
            pltpu.sync_copy(acc.at[pl.ds(base_row + _OPT * 8, 8)],
                            feat_o.at[pl.ds(cid * _N + base_row + _OPT * 8, 8)])

        @pl.when(sid < _GREM)
        def _drain_tail_w():
            pltpu.sync_copy(den0_sh.at[pl.ds(base_w + _WPT, 16)],
                            z_v.at[pl.ds(0, 16)])
            pltpu.sync_copy(z_v.at[pl.ds(0, 16)],
                            den_o.at[pl.ds(cid * _N + base_w + _WPT, 16)])
            if two_heads:
                pltpu.sync_copy(den1_sh.at[pl.ds(base_w + _WPT, 16)],
                                z_v.at[pl.ds(0, 16)])
                pltpu.sync_copy(z_v.at[pl.ds(0, 16)],
                                den_o.at[pl.ds((_NC + cid) * _N + base_w + _WPT, 16)])

    return pl.kernel(
        body,
        out_type=(
            jax.ShapeDtypeStruct((_NC * _N, _D), _f32),
            jax.ShapeDtypeStruct((nhd * _NC * _N,), _f32),
        ),
        mesh=_mesh,
        scratch_types=scratch,
        compiler_params=pltpu.CompilerParams(needs_layout_passes=False),
    )


_edge_pass2 = _make_edge_pass(True)
_edge_pass1 = _make_edge_pass(False)


def _den_col(dp, lo):
    return lax.dot_general(dp[lo:lo + _NC, :], jnp.ones((_NC, 1), _f32),
                           (((0,), (0,)), ((), ())),
                           preferred_element_type=_f32)


def _stage_a(x_ref, w1_ref, as1_ref, ad1_ref, wsk_ref, bsk_ref,
             h_ref, as0_o, as1_o, ad0_o, ad1_o, xskip_ref):
    x = x_ref[...]
    h = jnp.dot(x, w1_ref[...], preferred_element_type=_f32)
    att_s = as1_ref[...]
    att_d = ad1_ref[...]
    h0 = h[:, :_HPH]
    h1 = h[:, _HPH:]
    as0_o[...] = jnp.sum(h0 * att_s[0:1, :], axis=1, keepdims=True)
    as1_o[...] = jnp.sum(h1 * att_s[1:2, :], axis=1, keepdims=True)
    ad0_o[...] = jnp.sum(h0 * att_d[0:1, :], axis=1, keepdims=True)
    ad1_o[...] = jnp.sum(h1 * att_d[1:2, :], axis=1, keepdims=True)
    h_ref[...] = h
    xskip_ref[...] = (jnp.dot(x, wsk_ref[...], preferred_element_type=_f32)
                      + bsk_ref[...][None, :])


def _stage_c(part_ref, dp_ref, h1_ref, as0_ref, as1_ref, ad0_ref, ad1_ref,
             xskip_ref, b1_ref, g1_ref, bb1_ref, w2_ref, as2_ref, ad2_ref,
             wf_ref, bf_ref,
             h2_ref, a2s_ref, a2d_ref, xfin_ref):
    acc = part_ref[0] + part_ref[1]
    dp = dp_ref[0]
    e0 = jnp.exp(_lrelu(as0_ref[...] + ad0_ref[...]))
    e1 = jnp.exp(_lrelu(as1_ref[...] + ad1_ref[...]))
    h = h1_ref[...]
    num0 = acc[:, :_HPH] + h[:, :_HPH] * e0
    num1 = acc[:, _HPH:_D] + h[:, _HPH:_D] * e1
    den0 = _den_col(dp, 0) + e0
    den1 = _den_col(dp, _NC) + e1
    gat = jnp.concatenate([num0 / (den0 + 1e-16), num1 / (den1 + 1e-16)],
                          axis=1) + b1_ref[...][None, :]
    xhop = jnp.where(gat > 0.0, gat, jnp.exp(jnp.minimum(gat, 0.0)) - 1.0)
    xcomb = xhop + xskip_ref[...]
    m = jnp.mean(xcomb, axis=1, keepdims=True)
    v = jnp.mean((xcomb - m) ** 2, axis=1, keepdims=True)
    first = ((xcomb - m) / jnp.sqrt(v + 1e-5) * g1_ref[...][None, :]
             + bb1_ref[...][None, :])
    h2 = jnp.dot(first, w2_ref[...], preferred_element_type=_f32)
    a2s_ref[...] = jnp.sum(h2 * as2_ref[...], axis=1, keepdims=True)
    a2d_ref[...] = jnp.sum(h2 * ad2_ref[...], axis=1, keepdims=True)
    h2_ref[...] = h2
    xfin_ref[...] = (jnp.dot(first, wf_ref[...], preferred_element_type=_f32)
                     + bf_ref[...][None, :])


def _stage_e(part_ref, dp_ref, h2_ref, a2s_ref, a2d_ref, xfin_ref,
             b2_ref, g2_ref, bb2_ref, out_ref):
    acc = part_ref[0:_N, :] + part_ref[_N:2 * _N, :]
    e2 = jnp.exp(_lrelu(a2s_ref[...] + a2d_ref[...]))
    num = acc + h2_ref[...] * e2
    den = _den_col(dp_ref[0], 0) + e2
    x2 = num / (den + 1e-16) + b2_ref[...][None, :]
    y = x2 + xfin_ref[...]
    m = jnp.mean(y, axis=1, keepdims=True)
    v = jnp.mean((y - m) ** 2, axis=1, keepdims=True)
    out_ref[...] = ((y - m) / jnp.sqrt(v + 1e-5) * g2_ref[...][None, :]
                    + bb2_ref[...][None, :])


_stage_a_call = pl.pallas_call(
    _stage_a,
    out_shape=[
        jax.ShapeDtypeStruct((_N, _D), _f32),
        jax.ShapeDtypeStruct((_N, 1), _f32),
        jax.ShapeDtypeStruct((_N, 1), _f32),
        jax.ShapeDtypeStruct((_N, 1), _f32),
        jax.ShapeDtypeStruct((_N, 1), _f32),
        jax.ShapeDtypeStruct((_N, _D), _f32),
    ],
)
_BS = 2000
_stage_c_call = pl.pallas_call(
    _stage_c,
    grid=(_N // _BS,),
    in_specs=[
        pl.BlockSpec((2, _BS, _D), lambda i: (0, i, 0)),
        pl.BlockSpec((1, 2 * _NC, _BS), lambda i: (i, 0, 0)),
        pl.BlockSpec((_BS, _D), lambda i: (i, 0)),
        pl.BlockSpec((_BS, 1), lambda i: (i, 0)),
        pl.BlockSpec((_BS, 1), lambda i: (i, 0)),
        pl.BlockSpec((_BS, 1), lambda i: (i, 0)),
        pl.BlockSpec((_BS, 1), lambda i: (i, 0)),
        pl.BlockSpec((_BS, _D), lambda i: (i, 0)),
        pl.BlockSpec((_D,), lambda i: (0,)),
        pl.BlockSpec((_D,), lambda i: (0,)),
        pl.BlockSpec((_D,), lambda i: (0,)),
        pl.BlockSpec((_D, _D), lambda i: (0, 0)),
        pl.BlockSpec((1, _D), lambda i: (0, 0)),
        pl.BlockSpec((1, _D), lambda i: (0, 0)),
        pl.BlockSpec((_D, _D), lambda i: (0, 0)),
        pl.BlockSpec((_D,), lambda i: (0,)),
    ],
    out_specs=[
        pl.BlockSpec((_BS, _D), lambda i: (i, 0)),
        pl.BlockSpec((_BS, 1), lambda i: (i, 0)),
        pl.BlockSpec((_BS, 1), lambda i: (i, 0)),
        pl.BlockSpec((_BS, _D), lambda i: (i, 0)),
    ],
    out_shape=[
        jax.ShapeDtypeStruct((_N, _D), _f32),
        jax.ShapeDtypeStruct((_N, 1), _f32),
        jax.ShapeDtypeStruct((_N, 1), _f32),
        jax.ShapeDtypeStruct((_N, _D), _f32),
    ],
)
_stage_e_call = pl.pallas_call(
    _stage_e,
    out_shape=jax.ShapeDtypeStruct((_N, _D), _f32),
)


@jax.jit
def _run(x, edge_index, W1, att_src1, att_dst1, bias1, W2, att_src2,
         att_dst2, bias2, W_skip, b_skip, W_final, b_final,
         ln1_g, ln1_b, ln2_g, ln2_b):
    src = edge_index[0]
    dst2 = jnp.concatenate(
        [edge_index[1].reshape(_NBLK, _B),
         jnp.zeros((_DPAD - _NBLK, _B), jnp.int32)], axis=0)
    h1, as0, as1, ad0, ad1, xskip = _stage_a_call(
        x, W1, att_src1, att_dst1, W_skip, b_skip)
    part1, dp1 = _edge_pass2(h1, src, dst2,
                             as0.reshape(_N), ad0.reshape(_N),
                             as1.reshape(_N), ad1.reshape(_N))
    dp1 = jnp.swapaxes(dp1.reshape(2 * _NC, _N // _BS, _BS), 0, 1)
    part1 = part1.reshape(2, _N, _D)
    h2, a2s, a2d, xfin = _stage_c_call(
        part1, dp1, h1, as0, as1, ad0, ad1, xskip, bias1, ln1_g, ln1_b,
        W2, att_src2, att_dst2, W_final, b_final)
    part2, dp2 = _edge_pass1(h2, src, dst2, a2s.reshape(_N), a2d.reshape(_N))
    dp2 = dp2.reshape(1, _NC, _N)
    return _stage_e_call(part2, dp2, h2, a2s, a2d, xfin, bias2, ln2_g, ln2_b)


def kernel(x, edge_index, W1, att_src1, att_dst1, bias1, W2, att_src2,
           att_dst2, bias2, W_skip, b_skip, W_final, b_final,
           ln1_g, ln1_b, ln2_g, ln2_b):
    return _run(x, edge_index, W1, att_src1, att_dst1, bias1, W2, att_src2,
                att_dst2, bias2, W_skip, b_skip, W_final, b_final,
                ln1_g, ln1_b, ln2_g, ln2_b)


# speedup vs baseline: 110.7011x; 1.0127x over previous
import jax
import jax.numpy as jnp
from jax import lax
from jax.experimental import pallas as pl
from jax.experimental.pallas import tpu as pltpu
from jax.experimental.pallas import tpu_sc as plsc

_N = 10000
_E = 320000
_D = 128
_HPH = 64
_B = 128
_NC = 2
_NS = 16
_BPC = _E // _NC // _B
_BLK_LO = _BPC // _NS
_BLK_REM = _BPC % _NS
_OPT = _N // 8 // _NS
_OREM = (_N // 8) % _NS
_CH = 104
_NBLK = _E // _B
_DPAD = _NBLK + 4
_DSZ = 88
_G16 = _N // 16
_GPT = _G16 // _NS
_GREM = _G16 % _NS
_WPT = _GPT * 16
_f32 = jnp.float32


def _lrelu(a):
    return jnp.where(a >= 0.0, a, 0.2 * a)


_mesh = plsc.VectorSubcoreMesh(core_axis_name="c", subcore_axis_name="s")


def _make_edge_pass(two_heads):
    nhd = 2 if two_heads else 1
    bufset = (
        [pltpu.VMEM((_B,), jnp.int32)]
        + [pltpu.VMEM((_B, _D), _f32)]
        + [pltpu.VMEM((_B,), _f32) for _ in range(2 * nhd)]
        + [pltpu.VMEM((_B,), _f32) for _ in range(nhd)]
    )
    scratch = bufset + bufset + [
        pltpu.VMEM((_DSZ, _B), jnp.int32),
        pltpu.VMEM((640,), _f32),
    ]
    scratch += [pltpu.VMEM_SHARED((_N,), _f32) for _ in range(nhd)]
    scratch += [
        pltpu.VMEM_SHARED((_N, _D), _f32),
        pltpu.SemaphoreType.DMA,
        pltpu.SemaphoreType.DMA,
        pltpu.SemaphoreType.DMA,
        pltpu.SemaphoreType.DMA,
        pltpu.SemaphoreType.DMA,
        pltpu.SemaphoreType.DMA,
    ]

    def body(*refs):
        it = iter(refs)
        h_hbm = next(it)
        src_h = next(it)
        dst2_h = next(it)
        as0_h = next(it)
        ad0_h = next(it)
        as1_h = next(it) if two_heads else None
        ad1_h = next(it) if two_heads else None
        feat_o = next(it)
        den_o = next(it)

        def _take_set():
            s = {}
            s["src"] = next(it)
            s["rows"] = next(it)
            s["a0s"] = next(it)
            s["a0d"] = next(it)
            s["a1s"] = next(it) if two_heads else None
            s["a1d"] = next(it) if two_heads else None
            s["ex0"] = next(it)
            s["ex1"] = next(it) if two_heads else None
            return s

        bufA = _take_set()
        bufB = _take_set()
        dstall_v = next(it)
        z_v = next(it)
        den0_sh = next(it)
        den1_sh = next(it) if two_heads else None
        acc = next(it)
        bufA["gsem"] = next(it)
        bufB["gsem"] = next(it)
        bufA["ssem"] = next(it)
        bufB["ssem"] = next(it)
        bufA["isem"] = next(it)
        bufB["isem"] = next(it)
        rows_v = bufA["rows"]

        cid = lax.axis_index("c")
        sid = lax.axis_index("s")
        zero = jnp.zeros((16,), _f32)

        for i in range(640 // 16):
            z_v[pl.ds(i * 16, 16)] = zero

        def _zrow(r, c_):
            for c in range(_D // 16):
                rows_v[r, pl.ds(c * 16, 16)] = zero
            return c_

        lax.fori_loop(0, _CH, _zrow, 0)

        base_row = pl.multiple_of(8 * (_OPT * sid + jnp.minimum(sid, _OREM)), 8)
        for k in range(_OPT * 8 // _CH):
            pltpu.sync_copy(rows_v.at[pl.ds(0, _CH)],
                            acc.at[pl.ds(base_row + k * _CH, _CH)])

        @pl.when(sid < _OREM)
        def _init_tail():
            pltpu.sync_copy(rows_v.at[pl.ds(0, 8)],
                            acc.at[pl.ds(base_row + _OPT * 8, 8)])

        base_w = pl.multiple_of(16 * (_GPT * sid + jnp.minimum(sid, _GREM)), 16)
        pltpu.sync_copy(z_v.at[pl.ds(0, _WPT)],
                        den0_sh.at[pl.ds(base_w, _WPT)])
        if two_heads:
            pltpu.sync_copy(z_v.at[pl.ds(0, _WPT)],
                            den1_sh.at[pl.ds(base_w, _WPT)])

        @pl.when(sid < _GREM)
        def _init_tail_w():
            pltpu.sync_copy(z_v.at[pl.ds(0, 16)],
                            den0_sh.at[pl.ds(base_w + _WPT, 16)])
            if two_heads:
                pltpu.sync_copy(z_v.at[pl.ds(0, 16)],
                                den1_sh.at[pl.ds(base_w + _WPT, 16)])

        plsc.subcore_barrier()

        blk0 = cid * _BPC + _BLK_LO * sid + jnp.minimum(sid, _BLK_REM)
        last_blk = _E // _B - 1

        start8 = pl.multiple_of(
            jnp.minimum(8 * (blk0 // 8), _DPAD - _DSZ), 8)
        d0 = blk0 - start8
        pltpu.sync_copy(dst2_h.at[pl.ds(start8, _DSZ)], dstall_v)

        def _dstr(loc):
            return dstall_v.at[d0 + loc]

        def _fire_src(blk, buf):
            pltpu.async_copy(src_h.at[pl.ds(blk * _B, _B)], buf["src"],
                             buf["isem"])

        def _wait_src(buf):
            pltpu.make_async_copy(src_h.at[pl.ds(0, _B)], buf["src"],
                                  buf["isem"]).wait()

        def _fire(buf, loc):
            sem = buf["gsem"]
            srcr = buf["src"]
            dstr = _dstr(loc)
            pltpu.async_copy(h_hbm.at[srcr], buf["rows"], sem)
            pltpu.async_copy(as0_h.at[srcr], buf["a0s"], sem)
            pltpu.async_copy(ad0_h.at[dstr], buf["a0d"], sem)
            if two_heads:
                pltpu.async_copy(as1_h.at[srcr], buf["a1s"], sem)
                pltpu.async_copy(ad1_h.at[dstr], buf["a1d"], sem)

        def _drain(buf, loc):
            sem = buf["gsem"]
            srcr = buf["src"]
            dstr = _dstr(loc)
            pltpu.make_async_copy(as0_h.at[srcr], buf["a0s"], sem).wait()
            pltpu.make_async_copy(ad0_h.at[dstr], buf["a0d"], sem).wait()
            if two_heads:
                pltpu.make_async_copy(as1_h.at[srcr], buf["a1s"], sem).wait()
                pltpu.make_async_copy(ad1_h.at[dstr], buf["a1d"], sem).wait()
            pltpu.make_async_copy(h_hbm.at[srcr], buf["rows"], sem).wait()

        def _compute(buf):
            rows = buf["rows"]
            ex0_v = buf["ex0"]
            ex1_v = buf["ex1"]

            def _grp(g, c_):
                off = g * 16
                a0 = buf["a0s"][pl.ds(off, 16)] + buf["a0d"][pl.ds(off, 16)]
                ex0 = jnp.exp(_lrelu(a0))
                ex0_v[pl.ds(off, 16)] = ex0
                if two_heads:
                    a1 = buf["a1s"][pl.ds(off, 16)] + buf["a1d"][pl.ds(off, 16)]
                    ex1 = jnp.exp(_lrelu(a1))
                    ex1_v[pl.ds(off, 16)] = ex1
                else:
                    ex1 = ex0
                for j in range(16):
                    e = off + j
                    idx = jnp.full((16,), j, jnp.int32)
                    v0 = ex0.at[idx].get(mode="promise_in_bounds")
                    v1 = (ex1.at[idx].get(mode="promise_in_bounds")
                          if two_heads else v0)
                    for c in range(4):
                        rows[e, pl.ds(c * 16, 16)] = rows[e, pl.ds(c * 16, 16)] * v0
                    for c in range(4, 8):
                        rows[e, pl.ds(c * 16, 16)] = rows[e, pl.ds(c * 16, 16)] * v1
                return c_

            lax.fori_loop(0, _B // 16, _grp, 0)

        def _scatter(buf, loc):
            sem = buf["ssem"]
            dstr = _dstr(loc)
            pltpu.async_copy(buf["rows"], acc.at[dstr], sem, add=True)
            pltpu.async_copy(buf["ex0"], den0_sh.at[dstr], sem, add=True)
            if two_heads:
                pltpu.async_copy(buf["ex1"], den1_sh.at[dstr], sem, add=True)

        def _drain_scatter(buf, loc):
            sem = buf["ssem"]
            dstr = _dstr(loc)
            pltpu.make_async_copy(buf["rows"], acc.at[dstr], sem).wait()
            pltpu.make_async_copy(buf["ex0"], den0_sh.at[dstr], sem).wait()
            if two_heads:
                pltpu.make_async_copy(buf["ex1"], den1_sh.at[dstr], sem).wait()

        pltpu.sync_copy(src_h.at[pl.ds(blk0 * _B, _B)], bufA["src"])
        _fire(bufA, 0)
        _fire_src(blk0 + 1, bufB)

        def _pair(k, carry):
            bA = blk0 + 2 * k
            lA = 2 * k
            _wait_src(bufB)
            _fire(bufB, lA + 1)
            _drain(bufA, lA)
            _fire_src(jnp.minimum(bA + 2, last_blk), bufA)
            _compute(bufA)
            _scatter(bufA, lA)
            _drain(bufB, lA + 1)
            _fire_src(jnp.minimum(bA + 3, last_blk), bufB)
            _compute(bufB)
            _scatter(bufB, lA + 1)
            _drain_scatter(bufA, lA)
            _wait_src(bufA)
            _fire(bufA, jnp.minimum(lA + 2, _BLK_LO))
            _drain_scatter(bufB, lA + 1)
            return carry

        lax.fori_loop(0, _BLK_LO // 2, _pair, 0)

        _drain(bufA, _BLK_LO)
        _wait_src(bufB)

        @pl.when(sid < _BLK_REM)
        def _tail_block():
            _compute(bufA)
            _scatter(bufA, _BLK_LO)
            _drain_scatter(bufA, _BLK_LO)

        plsc.subcore_barrier()
        for k in range(_OPT * 8 // _CH):
            pltpu.sync_copy(acc.at[pl.ds(base_row + k * _CH, _CH)],
                            feat_o.at[pl.ds(cid * _N + base_row + k * _CH, _CH)])
        pltpu.sync_copy(den0_sh.at[pl.ds(base_w, _WPT)], z_v.at[pl.ds(0, _WPT)])
        pltpu.sync_copy(z_v.at[pl.ds(0, _WPT)],
                        den_o.at[pl.ds(cid * _N + base_w, _WPT)])
        if two_heads:
            pltpu.sync_copy(den1_sh.at[pl.ds(base_w, _WPT)],
                            z_v.at[pl.ds(0, _WPT)])
            pltpu.sync_copy(z_v.at[pl.ds(0, _WPT)],
                            den_o.at[pl.ds((_NC + cid) * _N + base_w, _WPT)])

        @pl.when(sid < _OREM)
        def _drain_tail():
            pltpu.sync_copy(acc.at[pl.ds(base_row + _OPT * 8, 8)],
                            feat_o.at[pl.ds(cid * _N + base_row + _OPT * 8, 8)])

        @pl.when(sid < _GREM)
        def _drain_tail_w():
            pltpu.sync_copy(den0_sh.at[pl.ds(base_w + _WPT, 16)],
                            z_v.at[pl.ds(0, 16)])
            pltpu.sync_copy(z_v.at[pl.ds(0, 16)],
                            den_o.at[pl.ds(cid * _N + base_w + _WPT, 16)])
            if two_heads:
                pltpu.sync_copy(den1_sh.at[pl.ds(base_w + _WPT, 16)],
                                z_v.at[pl.ds(0, 16)])
                pltpu.sync_copy(z_v.at[pl.ds(0, 16)],
                                den_o.at[pl.ds((_NC + cid) * _N + base_w + _WPT, 16)])

    return pl.kernel(
        body,
        out_type=(
            jax.ShapeDtypeStruct((_NC * _N, _D), _f32),
            jax.ShapeDtypeStruct((nhd * _NC * _N,), _f32),
        ),
        mesh=_mesh,
        scratch_types=scratch,
        compiler_params=pltpu.CompilerParams(needs_layout_passes=False),
    )


_edge_pass2 = _make_edge_pass(True)
_edge_pass1 = _make_edge_pass(False)


def _den_col(dp, lo):
    return lax.dot_general(dp[lo:lo + _NC, :], jnp.ones((_NC, 1), _f32),
                           (((0,), (0,)), ((), ())),
                           preferred_element_type=_f32)


def _stage_a(x_ref, w1_ref, as1_ref, ad1_ref, wsk_ref, bsk_ref,
             h_ref, as0_o, as1_o, ad0_o, ad1_o, xskip_ref):
    x = x_ref[...]
    h = jnp.dot(x, w1_ref[...], preferred_element_type=_f32)
    att_s = as1_ref[...]
    att_d = ad1_ref[...]
    h0 = h[:, :_HPH]
    h1 = h[:, _HPH:]
    as0_o[...] = jnp.sum(h0 * att_s[0:1, :], axis=1, keepdims=True)
    as1_o[...] = jnp.sum(h1 * att_s[1:2, :], axis=1, keepdims=True)
    ad0_o[...] = jnp.sum(h0 * att_d[0:1, :], axis=1, keepdims=True)
    ad1_o[...] = jnp.sum(h1 * att_d[1:2, :], axis=1, keepdims=True)
    h_ref[...] = h
    xskip_ref[...] = (jnp.dot(x, wsk_ref[...], preferred_element_type=_f32)
                      + bsk_ref[...][None, :])


def _stage_c(part_ref, dp_ref, h1_ref, as0_ref, as1_ref, ad0_ref, ad1_ref,
             xskip_ref, b1_ref, g1_ref, bb1_ref, w2_ref, as2_ref, ad2_ref,
             wf_ref, bf_ref,
             h2_ref, a2s_ref, a2d_ref, xfin_ref):
    acc = part_ref[0] + part_ref[1]
    dp = dp_ref[0]
    e0 = jnp.exp(_lrelu(as0_ref[...] + ad0_ref[...]))
    e1 = jnp.exp(_lrelu(as1_ref[...] + ad1_ref[...]))
    h = h1_ref[...]
    num0 = acc[:, :_HPH] + h[:, :_HPH] * e0
    num1 = acc[:, _HPH:_D] + h[:, _HPH:_D] * e1
    den0 = _den_col(dp, 0) + e0
    den1 = _den_col(dp, _NC) + e1
    gat = jnp.concatenate([num0 / (den0 + 1e-16), num1 / (den1 + 1e-16)],
                          axis=1) + b1_ref[...][None, :]
    xhop = jnp.where(gat > 0.0, gat, jnp.exp(jnp.minimum(gat, 0.0)) - 1.0)
    xcomb = xhop + xskip_ref[...]
    m = jnp.mean(xcomb, axis=1, keepdims=True)
    v = jnp.mean((xcomb - m) ** 2, axis=1, keepdims=True)
    first = ((xcomb - m) / jnp.sqrt(v + 1e-5) * g1_ref[...][None, :]
             + bb1_ref[...][None, :])
    h2 = jnp.dot(first, w2_ref[...], preferred_element_type=_f32)
    a2s_ref[...] = jnp.sum(h2 * as2_ref[...], axis=1, keepdims=True)
    a2d_ref[...] = jnp.sum(h2 * ad2_ref[...], axis=1, keepdims=True)
    h2_ref[...] = h2
    xfin_ref[...] = (jnp.dot(first, wf_ref[...], preferred_element_type=_f32)
                     + bf_ref[...][None, :])


def _stage_e(part_ref, dp_ref, h2_ref, a2s_ref, a2d_ref, xfin_ref,
             b2_ref, g2_ref, bb2_ref, out_ref):
    acc = part_ref[0:_N, :] + part_ref[_N:2 * _N, :]
    e2 = jnp.exp(_lrelu(a2s_ref[...] + a2d_ref[...]))
    num = acc + h2_ref[...] * e2
    den = _den_col(dp_ref[0], 0) + e2
    x2 = num / (den + 1e-16) + b2_ref[...][None, :]
    y = x2 + xfin_ref[...]
    m = jnp.mean(y, axis=1, keepdims=True)
    v = jnp.mean((y - m) ** 2, axis=1, keepdims=True)
    out_ref[...] = ((y - m) / jnp.sqrt(v + 1e-5) * g2_ref[...][None, :]
                    + bb2_ref[...][None, :])


_stage_a_call = pl.pallas_call(
    _stage_a,
    out_shape=[
        jax.ShapeDtypeStruct((_N, _D), _f32),
        jax.ShapeDtypeStruct((_N, 1), _f32),
        jax.ShapeDtypeStruct((_N, 1), _f32),
        jax.ShapeDtypeStruct((_N, 1), _f32),
        jax.ShapeDtypeStruct((_N, 1), _f32),
        jax.ShapeDtypeStruct((_N, _D), _f32),
    ],
)
_BS = 2000
_stage_c_call = pl.pallas_call(
    _stage_c,
    grid=(_N // _BS,),
    in_specs=[
        pl.BlockSpec((2, _BS, _D), lambda i: (0, i, 0)),
        pl.BlockSpec((1, 2 * _NC, _BS), lambda i: (i, 0, 0)),
        pl.BlockSpec((_BS, _D), lambda i: (i, 0)),
        pl.BlockSpec((_BS, 1), lambda i: (i, 0)),
        pl.BlockSpec((_BS, 1), lambda i: (i, 0)),
        pl.BlockSpec((_BS, 1), lambda i: (i, 0)),
        pl.BlockSpec((_BS, 1), lambda i: (i, 0)),
        pl.BlockSpec((_BS, _D), lambda i: (i, 0)),
        pl.BlockSpec((_D,), lambda i: (0,)),
        pl.BlockSpec((_D,), lambda i: (0,)),
        pl.BlockSpec((_D,), lambda i: (0,)),
        pl.BlockSpec((_D, _D), lambda i: (0, 0)),
        pl.BlockSpec((1, _D), lambda i: (0, 0)),
        pl.BlockSpec((1, _D), lambda i: (0, 0)),
        pl.BlockSpec((_D, _D), lambda i: (0, 0)),
        pl.BlockSpec((_D,), lambda i: (0,)),
    ],
    out_specs=[
        pl.BlockSpec((_BS, _D), lambda i: (i, 0)),
        pl.BlockSpec((_BS, 1), lambda i: (i, 0)),
        pl.BlockSpec((_BS, 1), lambda i: (i, 0)),
        pl.BlockSpec((_BS, _D), lambda i: (i, 0)),
    ],
    out_shape=[
        jax.ShapeDtypeStruct((_N, _D), _f32),
        jax.ShapeDtypeStruct((_N, 1), _f32),
        jax.ShapeDtypeStruct((_N, 1), _f32),
        jax.ShapeDtypeStruct((_N, _D), _f32),
    ],
)
_stage_e_call = pl.pallas_call(
    _stage_e,
    out_shape=jax.ShapeDtypeStruct((_N, _D), _f32),
)


@jax.jit
def _run(x, edge_index, W1, att_src1, att_dst1, bias1, W2, att_src2,
         att_dst2, bias2, W_skip, b_skip, W_final, b_final,
         ln1_g, ln1_b, ln2_g, ln2_b):
    src = edge_index[0]
    dst2 = jnp.concatenate(
        [edge_index[1].reshape(_NBLK, _B),
         jnp.zeros((_DPAD - _NBLK, _B), jnp.int32)], axis=0)
    h1, as0, as1, ad0, ad1, xskip = _stage_a_call(
        x, W1, att_src1, att_dst1, W_skip, b_skip)
    part1, dp1 = _edge_pass2(h1, src, dst2,
                             as0.reshape(_N), ad0.reshape(_N),
                             as1.reshape(_N), ad1.reshape(_N))
    dp1 = jnp.swapaxes(dp1.reshape(2 * _NC, _N // _BS, _BS), 0, 1)
    part1 = part1.reshape(2, _N, _D)
    h2, a2s, a2d, xfin = _stage_c_call(
        part1, dp1, h1, as0, as1, ad0, ad1, xskip, bias1, ln1_g, ln1_b,
        W2, att_src2, att_dst2, W_final, b_final)
    part2, dp2 = _edge_pass1(h2, src, dst2, a2s.reshape(_N), a2d.reshape(_N))
    dp2 = dp2.reshape(1, _NC, _N)
    return _stage_e_call(part2, dp2, h2, a2s, a2d, xfin, bias2, ln2_g, ln2_b)


def kernel(x, edge_index, W1, att_src1, att_dst1, bias1, W2, att_src2,
           att_dst2, bias2, W_skip, b_skip, W_final, b_final,
           ln1_g, ln1_b, ln2_g, ln2_b):
    return _run(x, edge_index, W1, att_src1, att_dst1, bias1, W2, att_src2,
                att_dst2, bias2, W_skip, b_skip, W_final, b_final,
                ln1_g, ln1_b, ln2_g, ln2_b)
